# Initial kernel scaffold; baseline (speedup 1.0000x reference)
import jax
import jax.numpy as jnp
from jax.experimental import pallas as pl


def kernel(indices, offsets, edge_index, ppi, self_w, emb_table, input_bias, W1, b1, W2, b2, Wout, bout):
    raise NotImplementedError("write your pallas kernel here")


# speedup vs baseline: 14.7501x; 14.7501x over previous
import functools

import jax
import jax.numpy as jnp
from jax import lax
from jax.experimental import pallas as pl
from jax.experimental.pallas import tpu as pltpu
from jax.experimental.pallas import tpu_sc as plsc

N_NODES = 10000
N_EDGES = 320000
NNZ = 300000
INPUT_SIZE = 40000
HIDDEN = 128
LABELS = 2000
HALF = 64

NC = 2
NS = 16
CHUNK = 128

KI = 147
NNZ_TILE = KI * CHUNK
NNZ_PAD = NNZ_TILE * NS
OFFS_PAD = 10016

KE = 160
SUP = 8
E_TILE = KE * CHUNK
E_PAD = E_TILE * NS
SUPE = SUP * CHUNK

ACC_ROWS = 10240
TRASH = N_NODES
ZSTRIPE = ACC_ROWS // NS
WSTRIPE = N_NODES // NS

_mesh = plsc.VectorSubcoreMesh(core_axis_name="c", subcore_axis_name="s")


def _zero_acc(gbuf, accs, s):
    z = jnp.zeros((16,), jnp.float32)

    def zrow(r, _):
        for u in range(4):
            gbuf[r, pl.ds(u * 16, 16)] = z
        return 0

    lax.fori_loop(0, CHUNK, zrow, 0)
    for acc in accs:
        for t in range(ZSTRIPE // CHUNK):
            pltpu.sync_copy(gbuf, acc.at[pl.ds(s * ZSTRIPE + t * CHUNK, CHUNK)])


@functools.partial(
    pl.kernel,
    out_type=jax.ShapeDtypeStruct((NC, N_NODES, HALF), jnp.float32),
    mesh=_mesh,
    compiler_params=pltpu.CompilerParams(use_tc_tiling_on_sc=False, needs_layout_passes=False),
    scratch_types=[
        pltpu.VMEM((NNZ_TILE,), jnp.int32),
        pltpu.VMEM((KI, CHUNK), jnp.int32),
        pltpu.VMEM((OFFS_PAD,), jnp.int32),
        pltpu.VMEM((CHUNK, HALF), jnp.float32),
        pltpu.VMEM((125, HALF), jnp.float32),
        pltpu.VMEM((HALF,), jnp.float32),
        pltpu.VMEM_SHARED((ACC_ROWS, HALF), jnp.float32),
    ],
)
def _emb_bag(emb_hbm, idx_hbm, offs_hbm, bias_hbm, out_hbm,
             idx_v, seg_v, offs_v, gbuf, obuf, bias_v, acc):
    c = lax.axis_index("c")
    s = lax.axis_index("s")

    _zero_acc(gbuf, (acc,), s)
    pltpu.sync_copy(offs_hbm, offs_v)
    pltpu.sync_copy(idx_hbm.at[pl.ds(s * NNZ_TILE, NNZ_TILE)], idx_v)
    pltpu.sync_copy(bias_hbm.at[pl.ds(c * HALF, HALF)], bias_v)
    plsc.subcore_barrier()

    pbase = s * NNZ_TILE

    def chunk_body(j, _):
        for u in range(8):
            p = pbase + j * CHUNK + u * 16 + lax.iota(jnp.int32, 16)
            lo = jnp.zeros((16,), jnp.int32)
            hi = jnp.full((16,), 10001, jnp.int32)

            def bs(_, lohi):
                lo_, hi_ = lohi
                mid = (lo_ + hi_) >> 1
                v = plsc.load_gather(offs_v, [mid])
                cge = v <= p
                return (jnp.where(cge, mid + 1, lo_), jnp.where(cge, hi_, mid))

            lo, hi = lax.fori_loop(0, 14, bs, (lo, hi))
            seg_v[j, pl.ds(u * 16, 16)] = lo - 1
        pltpu.sync_copy(emb_hbm.at[c].at[idx_v.at[pl.ds(j * CHUNK, CHUNK)]], gbuf)
        pltpu.sync_copy(gbuf, acc.at[seg_v.at[j]], add=True)
        return 0

    lax.fori_loop(0, KI, chunk_body, 0)
    plsc.subcore_barrier()

    for t in range(5):
        row0 = s * WSTRIPE + t * 125
        pltpu.sync_copy(acc.at[pl.ds(row0, 125)], obuf)

        def orow(r, _):
            for u in range(4):
                x = obuf[r, pl.ds(u * 16, 16)] + bias_v[pl.ds(u * 16, 16)]
                obuf[r, pl.ds(u * 16, 16)] = jnp.maximum(x, 0.0)
            return 0

        lax.fori_loop(0, 125, orow, 0)
        pltpu.sync_copy(obuf, out_hbm.at[c, pl.ds(row0, 125)])


@functools.partial(
    pl.kernel,
    out_type=(jax.ShapeDtypeStruct((NC, N_NODES, HALF), jnp.float32),
              jax.ShapeDtypeStruct((NC, N_NODES, HALF), jnp.float32)),
    mesh=_mesh,
    compiler_params=pltpu.CompilerParams(use_tc_tiling_on_sc=False, needs_layout_passes=False),
    scratch_types=[
        pltpu.VMEM((SUPE,), jnp.int32),
        pltpu.VMEM((SUP, CHUNK), jnp.int32),
        pltpu.VMEM((SUPE,), jnp.float32),
        pltpu.VMEM((SUPE,), jnp.float32),
        pltpu.VMEM((CHUNK, HALF), jnp.float32),
        pltpu.VMEM((CHUNK, HALF), jnp.float32),
        pltpu.VMEM((CHUNK, HALF), jnp.float32),
        pltpu.VMEM((125, HALF), jnp.float32),
        pltpu.VMEM_SHARED((ACC_ROWS, HALF), jnp.float32),
        pltpu.VMEM_SHARED((ACC_ROWS, HALF), jnp.float32),
    ],
)
def _gcn_scatter(h_hbm, src_hbm, dst_hbm, ppi_hbm, sw_hbm, outp_hbm, outr_hbm,
                 src_v, dst_v, ppi_v, sw_v, gbuf, sp, sr, obuf, acc_p, acc_r):
    c = lax.axis_index("c")
    s = lax.axis_index("s")

    _zero_acc(gbuf, (acc_p, acc_r), s)
    plsc.subcore_barrier()

    def sup_body(g, _):
        ebase = s * E_TILE + g * SUPE
        pltpu.sync_copy(src_hbm.at[pl.ds(ebase, SUPE)], src_v)
        pltpu.sync_copy(dst_hbm.at[pl.ds(s * KE + g * SUP, SUP)], dst_v)
        pltpu.sync_copy(ppi_hbm.at[pl.ds(ebase, SUPE)], ppi_v)
        pltpu.sync_copy(sw_hbm.at[pl.ds(ebase, SUPE)], sw_v)

        def chunk_body(j, _):
            pltpu.sync_copy(h_hbm.at[c].at[src_v.at[pl.ds(j * CHUNK, CHUNK)]],
                            gbuf)

            def row(r, _):
                e = jnp.full((16,), j * CHUNK + r, jnp.int32)
                wp = plsc.load_gather(ppi_v, [e])
                ws = plsc.load_gather(sw_v, [e])
                for u in range(4):
                    x = gbuf[r, pl.ds(u * 16, 16)]
                    sp[r, pl.ds(u * 16, 16)] = x * wp
                    sr[r, pl.ds(u * 16, 16)] = x * ws
                return 0

            lax.fori_loop(0, CHUNK, row, 0)
            pltpu.sync_copy(sp, acc_p.at[dst_v.at[j]], add=True)
            pltpu.sync_copy(sr, acc_r.at[dst_v.at[j]], add=True)
            return 0

        lax.fori_loop(0, SUP, chunk_body, 0)
        return 0

    lax.fori_loop(0, KE // SUP, sup_body, 0)
    plsc.subcore_barrier()

    for t in range(5):
        row0 = s * WSTRIPE + t * 125
        pltpu.sync_copy(acc_p.at[pl.ds(row0, 125)], obuf)
        pltpu.sync_copy(obuf, outp_hbm.at[c, pl.ds(row0, 125)])
        pltpu.sync_copy(acc_r.at[pl.ds(row0, 125)], obuf)
        pltpu.sync_copy(obuf, outr_hbm.at[c, pl.ds(row0, 125)])


_BM1 = 2000


def _node_update_body(p_ref, r_ref, w_ref, b_ref, o_ref):
    x = (jnp.dot(p_ref[0], w_ref[:HALF, :], preferred_element_type=jnp.float32)
         + jnp.dot(p_ref[1], w_ref[HALF:, :], preferred_element_type=jnp.float32))
    r = jnp.concatenate([r_ref[0], r_ref[1]], axis=-1)
    h = jnp.maximum(x + b_ref[0] + r, 0.0)
    o_ref[0] = h[:, :HALF]
    o_ref[1] = h[:, HALF:]


_node_update = pl.pallas_call(
    _node_update_body,
    grid=(N_NODES // _BM1,),
    in_specs=[
        pl.BlockSpec((NC, _BM1, HALF), lambda i: (0, i, 0)),
        pl.BlockSpec((NC, _BM1, HALF), lambda i: (0, i, 0)),
        pl.BlockSpec((HIDDEN, HIDDEN), lambda i: (0, 0)),
        pl.BlockSpec((1, HIDDEN), lambda i: (0, 0)),
    ],
    out_specs=pl.BlockSpec((NC, _BM1, HALF), lambda i: (0, i, 0)),
    out_shape=jax.ShapeDtypeStruct((NC, N_NODES, HALF), jnp.float32),
)

_BM2 = 1000


def _final_body(p_ref, r_ref, w_ref, b_ref, wo_ref, bo_ref, o_ref):
    x = (jnp.dot(p_ref[0], w_ref[:HALF, :], preferred_element_type=jnp.float32)
         + jnp.dot(p_ref[1], w_ref[HALF:, :], preferred_element_type=jnp.float32))
    r = jnp.concatenate([r_ref[0], r_ref[1]], axis=-1)
    h = jnp.maximum(x + b_ref[0] + r, 0.0)
    o_ref[...] = jnp.dot(h, wo_ref[...], preferred_element_type=jnp.float32) + bo_ref[0]


_final_mm = pl.pallas_call(
    _final_body,
    grid=(N_NODES // _BM2,),
    in_specs=[
        pl.BlockSpec((NC, _BM2, HALF), lambda i: (0, i, 0)),
        pl.BlockSpec((NC, _BM2, HALF), lambda i: (0, i, 0)),
        pl.BlockSpec((HIDDEN, HIDDEN), lambda i: (0, 0)),
        pl.BlockSpec((1, HIDDEN), lambda i: (0, 0)),
        pl.BlockSpec((HIDDEN, LABELS), lambda i: (0, 0)),
        pl.BlockSpec((1, LABELS), lambda i: (0, 0)),
    ],
    out_specs=pl.BlockSpec((_BM2, LABELS), lambda i: (i, 0)),
    out_shape=jax.ShapeDtypeStruct((N_NODES, LABELS), jnp.float32),
)


def kernel(indices, offsets, edge_index, ppi, self_w, emb_table, input_bias,
           W1, b1, W2, b2, Wout, bout):
    emb_split = emb_table.reshape(INPUT_SIZE, NC, HALF).transpose(1, 0, 2)
    idx_pad = jnp.pad(indices, (0, NNZ_PAD - NNZ))
    offs_pad = jnp.pad(offsets, (0, OFFS_PAD - (N_NODES + 1)),
                       constant_values=NNZ)
    src_pad = jnp.pad(edge_index[0], (0, E_PAD - N_EDGES))
    dst2d = jnp.pad(edge_index[1], (0, E_PAD - N_EDGES),
                    constant_values=TRASH).reshape(E_PAD // CHUNK, CHUNK)
    ppi_pad = jnp.pad(ppi, (0, E_PAD - N_EDGES))
    sw_pad = jnp.pad(self_w, (0, E_PAD - N_EDGES))

    h = _emb_bag(emb_split, idx_pad, offs_pad, input_bias)
    pp, rr = _gcn_scatter(h, src_pad, dst2d, ppi_pad, sw_pad)
    h = _node_update(pp, rr, W1.T, b1.reshape(1, HIDDEN))
    pp, rr = _gcn_scatter(h, src_pad, dst2d, ppi_pad, sw_pad)
    return _final_mm(pp, rr, W2.T, b2.reshape(1, HIDDEN),
                     Wout.T, bout.reshape(1, LABELS))


# speedup vs baseline: 17.8803x; 1.2122x over previous
import functools

import jax
import jax.numpy as jnp
from jax import lax
from jax.experimental import pallas as pl
from jax.experimental.pallas import tpu as pltpu
from jax.experimental.pallas import tpu_sc as plsc

N_NODES = 10000
N_EDGES = 320000
NNZ = 300000
INPUT_SIZE = 40000
HIDDEN = 128
LABELS = 2000
HALF = 64

NC = 2
NS = 16

CH_E = 128
KI = 148
NNZ_TILE = KI * CH_E
NNZ_PAD = NNZ_TILE * NS
OFFS_PAD = 10016

CH_G = 64
SUP = 16
SUPE = SUP * CH_G
NSUP = 20
E_TILE = NSUP * SUPE
E_PAD = E_TILE * NS
E_PADX = E_PAD + 2 * SUPE

ACC_ROWS = 10240
TRASH = N_NODES
WSTRIPE = N_NODES // NS

_mesh = plsc.VectorSubcoreMesh(core_axis_name="c", subcore_axis_name="s")
_params = pltpu.CompilerParams(use_tc_tiling_on_sc=False,
                               needs_layout_passes=False)


def _zero_rows(buf, n):
    z = jnp.zeros((16,), jnp.float32)
    cols = buf.shape[1]

    def zrow(r, _):
        for u in range(cols // 16):
            buf[r, pl.ds(u * 16, 16)] = z
        return 0

    lax.fori_loop(0, n, zrow, 0)


@functools.partial(
    pl.kernel,
    out_type=jax.ShapeDtypeStruct((NC, N_NODES, HALF), jnp.float32),
    mesh=_mesh,
    compiler_params=_params,
    scratch_types=[
        pltpu.VMEM((NNZ_TILE,), jnp.int32),
        pltpu.VMEM((4, CH_E), jnp.int32),
        pltpu.VMEM((OFFS_PAD,), jnp.int32),
        [pltpu.VMEM((CH_E, HALF), jnp.float32) for _ in range(4)],
        pltpu.VMEM((125, HALF), jnp.float32),
        pltpu.VMEM((HALF,), jnp.float32),
        pltpu.VMEM_SHARED((ACC_ROWS, HALF), jnp.float32),
        [pltpu.SemaphoreType.DMA for _ in range(4)],
        [pltpu.SemaphoreType.DMA for _ in range(4)],
    ],
)
def _emb_bag(emb_hbm, idx_hbm, offs_hbm, bias_hbm, out_hbm,
             idx_v, seg_v, offs_v, gbufs, obuf, bias_v, acc, gsems, ssems):
    c = lax.axis_index("c")
    s = lax.axis_index("s")

    _zero_rows(gbufs[0], CH_E)
    _zero_rows(gbufs[1], CH_E)
    _zero_rows(gbufs[2], CH_E)
    _zero_rows(gbufs[3], CH_E)
    for t in range(5):
        pltpu.sync_copy(gbufs[t % 4], acc.at[pl.ds(s * 640 + t * CH_E, CH_E)])
    pltpu.sync_copy(offs_hbm, offs_v)
    pltpu.sync_copy(idx_hbm.at[pl.ds(s * NNZ_TILE, NNZ_TILE)], idx_v)
    pltpu.sync_copy(bias_hbm.at[pl.ds(c * HALF, HALF)], bias_v)
    plsc.subcore_barrier()
    for b in range(4):
        pltpu.async_copy(gbufs[b], acc.at[pl.ds(10048, CH_E)], ssems[b])

    pbase = s * NNZ_TILE
    emb_c = emb_hbm.at[c]

    def chunk_group(k4, _):
        for b in range(4):
            j = k4 * 4 + b
            pltpu.make_async_copy(
                gbufs[b], acc.at[pl.ds(0, CH_E)], ssems[b]).wait()
            pltpu.async_copy(emb_c.at[idx_v.at[pl.ds(j * CH_E, CH_E)]],
                             gbufs[b], gsems[b])
            for u in range(8):
                p = pbase + j * CH_E + u * 16 + lax.iota(jnp.int32, 16)
                lo = jnp.zeros((16,), jnp.int32)
                hi = jnp.full((16,), 10001, jnp.int32)

                def bs(_, lohi):
                    lo_, hi_ = lohi
                    mid = (lo_ + hi_) >> 1
                    v = plsc.load_gather(offs_v, [mid])
                    cge = v <= p
                    return (jnp.where(cge, mid + 1, lo_),
                            jnp.where(cge, hi_, mid))

                lo, hi = lax.fori_loop(0, 14, bs, (lo, hi))
                seg_v[b, pl.ds(u * 16, 16)] = lo - 1
            pltpu.make_async_copy(
                emb_c.at[idx_v.at[pl.ds(j * CH_E, CH_E)]],
                gbufs[b], gsems[b]).wait()
            pltpu.async_copy(gbufs[b], acc.at[seg_v.at[b]], ssems[b],
                             add=True)
        return 0

    lax.fori_loop(0, KI // 4, chunk_group, 0)
    for b in range(4):
        pltpu.make_async_copy(
            gbufs[b], acc.at[pl.ds(0, CH_E)], ssems[b]).wait()
    plsc.subcore_barrier()

    for t in range(5):
        row0 = s * WSTRIPE + t * 125
        pltpu.sync_copy(acc.at[pl.ds(row0, 125)], obuf)

        def orow(r, _):
            for u in range(4):
                x = obuf[r, pl.ds(u * 16, 16)] + bias_v[pl.ds(u * 16, 16)]
                obuf[r, pl.ds(u * 16, 16)] = jnp.maximum(x, 0.0)
            return 0

        lax.fori_loop(0, 125, orow, 0)
        pltpu.sync_copy(obuf, out_hbm.at[c, pl.ds(row0, 125)])


@functools.partial(
    pl.kernel,
    out_type=(jax.ShapeDtypeStruct((NC, N_NODES, HALF), jnp.float32),
              jax.ShapeDtypeStruct((NC, N_NODES, HALF), jnp.float32)),
    mesh=_mesh,
    compiler_params=_params,
    scratch_types=[
        pltpu.VMEM((2, SUPE), jnp.int32),
        pltpu.VMEM((2, SUP, CH_G), jnp.int32),
        pltpu.VMEM((2, SUPE), jnp.float32),
        pltpu.VMEM((2, SUPE), jnp.float32),
        [pltpu.VMEM((CH_G, HALF), jnp.float32) for _ in range(2)],
        [pltpu.VMEM((CH_G, HIDDEN), jnp.float32) for _ in range(2)],
        pltpu.VMEM((125, HALF), jnp.float32),
        pltpu.VMEM_SHARED((ACC_ROWS, HIDDEN), jnp.float32),
        [pltpu.SemaphoreType.DMA for _ in range(2)],
        [pltpu.SemaphoreType.DMA for _ in range(2)],
        pltpu.SemaphoreType.DMA,
    ],
)
def _gcn_scatter(h_hbm, src_hbm, dst_hbm, ppi_hbm, sw_hbm, outp_hbm, outr_hbm,
                 src_v, dst_v, ppi_v, sw_v, gbufs, sbufs, obuf, acc,
                 gsems, ssems, msem):
    c = lax.axis_index("c")
    s = lax.axis_index("s")
    h_c = h_hbm.at[c]
    ebase = s * E_TILE
    dbase = s * (E_TILE // CH_G)

    def _meta_fire(g, slot):
        pltpu.async_copy(src_hbm.at[pl.ds(ebase + g * SUPE, SUPE)],
                         src_v.at[slot], msem)
        pltpu.async_copy(dst_hbm.at[pl.ds(dbase + g * SUP, SUP)],
                         dst_v.at[slot], msem)
        pltpu.async_copy(ppi_hbm.at[pl.ds(ebase + g * SUPE, SUPE)],
                         ppi_v.at[slot], msem)
        pltpu.async_copy(sw_hbm.at[pl.ds(ebase + g * SUPE, SUPE)],
                         sw_v.at[slot], msem)

    def _meta_wait(slot):
        pltpu.make_async_copy(src_hbm.at[pl.ds(0, SUPE)],
                              src_v.at[slot], msem).wait()
        pltpu.make_async_copy(dst_hbm.at[pl.ds(0, SUP)],
                              dst_v.at[slot], msem).wait()
        pltpu.make_async_copy(ppi_hbm.at[pl.ds(0, SUPE)],
                              ppi_v.at[slot], msem).wait()
        pltpu.make_async_copy(sw_hbm.at[pl.ds(0, SUPE)],
                              sw_v.at[slot], msem).wait()

    def _gather_fire(slot, jj, b):
        pltpu.async_copy(
            h_c.at[src_v.at[slot].at[pl.ds(jj * CH_G, CH_G)]],
            gbufs[b], gsems[b])

    def _gather_wait(b):
        pltpu.make_async_copy(h_c.at[src_v.at[0].at[pl.ds(0, CH_G)]],
                              gbufs[b], gsems[b]).wait()

    def _scatter_wait(b):
        pltpu.make_async_copy(sbufs[b], acc.at[pl.ds(0, CH_G)],
                              ssems[b]).wait()

    _zero_rows(sbufs[0], CH_G)
    _zero_rows(sbufs[1], CH_G)
    for t in range(10):
        pltpu.sync_copy(sbufs[t % 2],
                        acc.at[pl.ds(s * 640 + t * CH_G, CH_G)])
    plsc.subcore_barrier()
    for b in range(2):
        pltpu.async_copy(sbufs[b], acc.at[pl.ds(10048 + b * CH_G, CH_G)],
                         ssems[b])

    _meta_fire(0, 0)
    _meta_wait(0)
    _gather_fire(0, 0, 0)

    def pair_body(i, _):
        for m in range(2):
            g = i * 2 + m
            for jj in range(SUP):
                b = jj % 2
                _gather_wait(b)
                if jj < SUP - 1:
                    _gather_fire(m, jj + 1, b ^ 1)
                else:
                    _meta_wait(m ^ 1)
                    _gather_fire(m ^ 1, 0, b ^ 1)
                _scatter_wait(b)
                if jj == 1:
                    _meta_fire(g + 1, m ^ 1)

                def row(r, _):
                    e = jnp.full((16,), jj * CH_G + r, jnp.int32)
                    sm = jnp.full((16,), m, jnp.int32)
                    wp = plsc.load_gather(ppi_v, [sm, e])
                    ws = plsc.load_gather(sw_v, [sm, e])
                    for u in range(4):
                        x = gbufs[b][r, pl.ds(u * 16, 16)]
                        sbufs[b][r, pl.ds(u * 16, 16)] = x * wp
                        sbufs[b][r, pl.ds(HALF + u * 16, 16)] = x * ws
                    return 0

                lax.fori_loop(0, CH_G, row, 0)
                pltpu.async_copy(sbufs[b], acc.at[dst_v.at[m, jj]],
                                 ssems[b], add=True)
        return 0

    lax.fori_loop(0, NSUP // 2, pair_body, 0)

    _gather_wait(0)
    _scatter_wait(0)
    _scatter_wait(1)
    plsc.subcore_barrier()

    for t in range(5):
        row0 = s * WSTRIPE + t * 125
        pltpu.sync_copy(acc.at[pl.ds(row0, 125), pl.ds(0, HALF)], obuf)
        pltpu.sync_copy(obuf, outp_hbm.at[c, pl.ds(row0, 125)])
        pltpu.sync_copy(acc.at[pl.ds(row0, 125), pl.ds(HALF, HALF)], obuf)
        pltpu.sync_copy(obuf, outr_hbm.at[c, pl.ds(row0, 125)])


_BM1 = 2000


def _node_update_body(p_ref, r_ref, w_ref, b_ref, o_ref):
    x = (jnp.dot(p_ref[0], w_ref[:HALF, :], preferred_element_type=jnp.float32)
         + jnp.dot(p_ref[1], w_ref[HALF:, :], preferred_element_type=jnp.float32))
    r = jnp.concatenate([r_ref[0], r_ref[1]], axis=-1)
    h = jnp.maximum(x + b_ref[0] + r, 0.0)
    o_ref[0] = h[:, :HALF]
    o_ref[1] = h[:, HALF:]


_node_update = pl.pallas_call(
    _node_update_body,
    grid=(N_NODES // _BM1,),
    in_specs=[
        pl.BlockSpec((NC, _BM1, HALF), lambda i: (0, i, 0)),
        pl.BlockSpec((NC, _BM1, HALF), lambda i: (0, i, 0)),
        pl.BlockSpec((HIDDEN, HIDDEN), lambda i: (0, 0)),
        pl.BlockSpec((1, HIDDEN), lambda i: (0, 0)),
    ],
    out_specs=pl.BlockSpec((NC, _BM1, HALF), lambda i: (0, i, 0)),
    out_shape=jax.ShapeDtypeStruct((NC, N_NODES, HALF), jnp.float32),
)

_BM2 = 1000


def _final_body(p_ref, r_ref, w_ref, b_ref, wo_ref, bo_ref, o_ref):
    x = (jnp.dot(p_ref[0], w_ref[:HALF, :], preferred_element_type=jnp.float32)
         + jnp.dot(p_ref[1], w_ref[HALF:, :], preferred_element_type=jnp.float32))
    r = jnp.concatenate([r_ref[0], r_ref[1]], axis=-1)
    h = jnp.maximum(x + b_ref[0] + r, 0.0)
    o_ref[...] = jnp.dot(h, wo_ref[...], preferred_element_type=jnp.float32) + bo_ref[0]


_final_mm = pl.pallas_call(
    _final_body,
    grid=(N_NODES // _BM2,),
    in_specs=[
        pl.BlockSpec((NC, _BM2, HALF), lambda i: (0, i, 0)),
        pl.BlockSpec((NC, _BM2, HALF), lambda i: (0, i, 0)),
        pl.BlockSpec((HIDDEN, HIDDEN), lambda i: (0, 0)),
        pl.BlockSpec((1, HIDDEN), lambda i: (0, 0)),
        pl.BlockSpec((HIDDEN, LABELS), lambda i: (0, 0)),
        pl.BlockSpec((1, LABELS), lambda i: (0, 0)),
    ],
    out_specs=pl.BlockSpec((_BM2, LABELS), lambda i: (i, 0)),
    out_shape=jax.ShapeDtypeStruct((N_NODES, LABELS), jnp.float32),
)


def kernel(indices, offsets, edge_index, ppi, self_w, emb_table, input_bias,
           W1, b1, W2, b2, Wout, bout):
    emb_split = emb_table.reshape(INPUT_SIZE, NC, HALF).transpose(1, 0, 2)
    idx_pad = jnp.pad(indices, (0, NNZ_PAD - NNZ))
    offs_pad = jnp.pad(offsets, (0, OFFS_PAD - (N_NODES + 1)),
                       constant_values=NNZ)
    src_pad = jnp.pad(edge_index[0], (0, E_PADX - N_EDGES))
    dst2d = jnp.pad(edge_index[1], (0, E_PADX - N_EDGES),
                    constant_values=TRASH).reshape(E_PADX // CH_G, CH_G)
    ppi_pad = jnp.pad(ppi, (0, E_PADX - N_EDGES))
    sw_pad = jnp.pad(self_w, (0, E_PADX - N_EDGES))

    h = _emb_bag(emb_split, idx_pad, offs_pad, input_bias)
    pp, rr = _gcn_scatter(h, src_pad, dst2d, ppi_pad, sw_pad)
    h = _node_update(pp, rr, W1.T, b1.reshape(1, HIDDEN))
    pp, rr = _gcn_scatter(h, src_pad, dst2d, ppi_pad, sw_pad)
    return _final_mm(pp, rr, W2.T, b2.reshape(1, HIDDEN),
                     Wout.T, bout.reshape(1, LABELS))


# speedup vs baseline: 21.3468x; 1.1939x over previous
import functools

import jax
import jax.numpy as jnp
from jax import lax
from jax.experimental import pallas as pl
from jax.experimental.pallas import tpu as pltpu
from jax.experimental.pallas import tpu_sc as plsc

N_NODES = 10000
N_EDGES = 320000
NNZ = 300000
INPUT_SIZE = 40000
HIDDEN = 128
LABELS = 2000
HALF = 64

NC = 2
NS = 16

CH_E = 128
KI = 148
NNZ_TILE = KI * CH_E
NNZ_PAD = NNZ_TILE * NS
OFFS_PAD = 10016

CH_G = 64
SUP = 16
SUPE = SUP * CH_G
NSUP = 20
E_TILE = NSUP * SUPE
E_PAD = E_TILE * NS
E_PADX = E_PAD + 2 * SUPE

ACC_ROWS = 10240
TRASH = N_NODES
WSTRIPE = N_NODES // NS

_mesh = plsc.VectorSubcoreMesh(core_axis_name="c", subcore_axis_name="s")
_params = pltpu.CompilerParams(use_tc_tiling_on_sc=False,
                               needs_layout_passes=False)


def _zero_rows(buf, n):
    z = jnp.zeros((16,), jnp.float32)
    cols = buf.shape[1]

    def zrow(r, _):
        for u in range(cols // 16):
            buf[r, pl.ds(u * 16, 16)] = z
        return 0

    lax.fori_loop(0, n, zrow, 0)


@functools.partial(
    pl.kernel,
    out_type=jax.ShapeDtypeStruct((NC, N_NODES, HALF), jnp.float32),
    mesh=_mesh,
    compiler_params=_params,
    scratch_types=[
        pltpu.VMEM((NNZ_TILE,), jnp.int32),
        pltpu.VMEM((4, CH_E), jnp.int32),
        pltpu.VMEM((OFFS_PAD,), jnp.int32),
        [pltpu.VMEM((CH_E, HALF), jnp.float32) for _ in range(4)],
        pltpu.VMEM((125, HALF), jnp.float32),
        pltpu.VMEM((HALF,), jnp.float32),
        pltpu.VMEM_SHARED((ACC_ROWS, HALF), jnp.float32),
        [pltpu.SemaphoreType.DMA for _ in range(4)],
        [pltpu.SemaphoreType.DMA for _ in range(4)],
    ],
)
def _emb_bag(emb_hbm, idx_hbm, offs_hbm, bias_hbm, out_hbm,
             idx_v, seg_v, offs_v, gbufs, obuf, bias_v, acc, gsems, ssems):
    c = lax.axis_index("c")
    s = lax.axis_index("s")

    _zero_rows(gbufs[0], CH_E)
    _zero_rows(gbufs[1], CH_E)
    _zero_rows(gbufs[2], CH_E)
    _zero_rows(gbufs[3], CH_E)
    for t in range(5):
        pltpu.sync_copy(gbufs[t % 4], acc.at[pl.ds(s * 640 + t * CH_E, CH_E)])
    pltpu.sync_copy(offs_hbm, offs_v)
    pltpu.sync_copy(idx_hbm.at[pl.ds(s * NNZ_TILE, NNZ_TILE)], idx_v)
    pltpu.sync_copy(bias_hbm.at[pl.ds(c * HALF, HALF)], bias_v)
    plsc.subcore_barrier()
    for b in range(4):
        pltpu.async_copy(gbufs[b], acc.at[pl.ds(10048, CH_E)], ssems[b])

    pbase = s * NNZ_TILE
    emb_c = emb_hbm.at[c]

    def chunk_group(k4, _):
        for b in range(4):
            j = k4 * 4 + b
            pltpu.make_async_copy(
                gbufs[b], acc.at[pl.ds(0, CH_E)], ssems[b]).wait()
            pltpu.async_copy(emb_c.at[idx_v.at[pl.ds(j * CH_E, CH_E)]],
                             gbufs[b], gsems[b])
            ps = tuple(pbase + j * CH_E + u * 16 + lax.iota(jnp.int32, 16)
                       for u in range(8))
            lo0 = tuple(jnp.zeros((16,), jnp.int32) for _ in range(8))
            hi0 = tuple(jnp.full((16,), 10001, jnp.int32) for _ in range(8))

            def bs(_, lohi):
                los, his = lohi
                nlo, nhi = [], []
                for u in range(8):
                    mid = (los[u] + his[u]) >> 1
                    v = plsc.load_gather(offs_v, [mid])
                    cge = v <= ps[u]
                    nlo.append(jnp.where(cge, mid + 1, los[u]))
                    nhi.append(jnp.where(cge, his[u], mid))
                return (tuple(nlo), tuple(nhi))

            los, his = lax.fori_loop(0, 14, bs, (lo0, hi0))
            for u in range(8):
                seg_v[b, pl.ds(u * 16, 16)] = los[u] - 1
            pltpu.make_async_copy(
                emb_c.at[idx_v.at[pl.ds(j * CH_E, CH_E)]],
                gbufs[b], gsems[b]).wait()
            pltpu.async_copy(gbufs[b], acc.at[seg_v.at[b]], ssems[b],
                             add=True)
        return 0

    lax.fori_loop(0, KI // 4, chunk_group, 0)
    for b in range(4):
        pltpu.make_async_copy(
            gbufs[b], acc.at[pl.ds(0, CH_E)], ssems[b]).wait()
    plsc.subcore_barrier()

    for t in range(5):
        row0 = s * WSTRIPE + t * 125
        pltpu.sync_copy(acc.at[pl.ds(row0, 125)], obuf)

        @plsc.parallel_loop(0, 125, unroll=4)
        def orow(r):
            for u in range(4):
                x = obuf[r, pl.ds(u * 16, 16)] + bias_v[pl.ds(u * 16, 16)]
                obuf[r, pl.ds(u * 16, 16)] = jnp.maximum(x, 0.0)
        pltpu.sync_copy(obuf, out_hbm.at[c, pl.ds(row0, 125)])


@functools.partial(
    pl.kernel,
    out_type=(jax.ShapeDtypeStruct((NC, N_NODES, HALF), jnp.float32),
              jax.ShapeDtypeStruct((NC, N_NODES, HALF), jnp.float32)),
    mesh=_mesh,
    compiler_params=_params,
    scratch_types=[
        pltpu.VMEM((2, SUPE), jnp.int32),
        pltpu.VMEM((2, SUP, CH_G), jnp.int32),
        pltpu.VMEM((2, SUPE), jnp.float32),
        pltpu.VMEM((2, SUPE), jnp.float32),
        [pltpu.VMEM((CH_G, HALF), jnp.float32) for _ in range(2)],
        [pltpu.VMEM((CH_G, HIDDEN), jnp.float32) for _ in range(2)],
        pltpu.VMEM((125, HALF), jnp.float32),
        pltpu.VMEM_SHARED((ACC_ROWS, HIDDEN), jnp.float32),
        [pltpu.SemaphoreType.DMA for _ in range(2)],
        [pltpu.SemaphoreType.DMA for _ in range(2)],
        pltpu.SemaphoreType.DMA,
    ],
)
def _gcn_scatter(h_hbm, src_hbm, dst_hbm, ppi_hbm, sw_hbm, outp_hbm, outr_hbm,
                 src_v, dst_v, ppi_v, sw_v, gbufs, sbufs, obuf, acc,
                 gsems, ssems, msem):
    c = lax.axis_index("c")
    s = lax.axis_index("s")
    h_c = h_hbm.at[c]
    ebase = s * E_TILE
    dbase = s * (E_TILE // CH_G)

    def _meta_fire(g, slot):
        pltpu.async_copy(src_hbm.at[pl.ds(ebase + g * SUPE, SUPE)],
                         src_v.at[slot], msem)
        pltpu.async_copy(dst_hbm.at[pl.ds(dbase + g * SUP, SUP)],
                         dst_v.at[slot], msem)
        pltpu.async_copy(ppi_hbm.at[pl.ds(ebase + g * SUPE, SUPE)],
                         ppi_v.at[slot], msem)
        pltpu.async_copy(sw_hbm.at[pl.ds(ebase + g * SUPE, SUPE)],
                         sw_v.at[slot], msem)

    def _meta_wait(slot):
        pltpu.make_async_copy(src_hbm.at[pl.ds(0, SUPE)],
                              src_v.at[slot], msem).wait()
        pltpu.make_async_copy(dst_hbm.at[pl.ds(0, SUP)],
                              dst_v.at[slot], msem).wait()
        pltpu.make_async_copy(ppi_hbm.at[pl.ds(0, SUPE)],
                              ppi_v.at[slot], msem).wait()
        pltpu.make_async_copy(sw_hbm.at[pl.ds(0, SUPE)],
                              sw_v.at[slot], msem).wait()

    def _gather_fire(slot, jj, b):
        pltpu.async_copy(
            h_c.at[src_v.at[slot].at[pl.ds(jj * CH_G, CH_G)]],
            gbufs[b], gsems[b])

    def _gather_wait(b):
        pltpu.make_async_copy(h_c.at[src_v.at[0].at[pl.ds(0, CH_G)]],
                              gbufs[b], gsems[b]).wait()

    def _scatter_wait(b):
        pltpu.make_async_copy(sbufs[b], acc.at[pl.ds(0, CH_G)],
                              ssems[b]).wait()

    _zero_rows(sbufs[0], CH_G)
    _zero_rows(sbufs[1], CH_G)
    for t in range(10):
        pltpu.sync_copy(sbufs[t % 2],
                        acc.at[pl.ds(s * 640 + t * CH_G, CH_G)])
    plsc.subcore_barrier()
    for b in range(2):
        pltpu.async_copy(sbufs[b], acc.at[pl.ds(10048 + b * CH_G, CH_G)],
                         ssems[b])

    _meta_fire(0, 0)
    _meta_wait(0)
    _gather_fire(0, 0, 0)

    def pair_body(i, _):
        for m in range(2):
            g = i * 2 + m
            for jj in range(SUP):
                b = jj % 2
                _gather_wait(b)
                if jj < SUP - 1:
                    _gather_fire(m, jj + 1, b ^ 1)
                else:
                    _meta_wait(m ^ 1)
                    _gather_fire(m ^ 1, 0, b ^ 1)
                _scatter_wait(b)
                if jj == 1:
                    _meta_fire(g + 1, m ^ 1)

                @plsc.parallel_loop(0, CH_G, unroll=4)
                def row(r):
                    e = jnp.full((16,), jj * CH_G, jnp.int32) + r
                    sm = jnp.full((16,), m, jnp.int32)
                    wp = plsc.load_gather(ppi_v, [sm, e])
                    ws = plsc.load_gather(sw_v, [sm, e])
                    for u in range(4):
                        x = gbufs[b][r, pl.ds(u * 16, 16)]
                        sbufs[b][r, pl.ds(u * 16, 16)] = x * wp
                        sbufs[b][r, pl.ds(HALF + u * 16, 16)] = x * ws
                pltpu.async_copy(sbufs[b], acc.at[dst_v.at[m, jj]],
                                 ssems[b], add=True)
        return 0

    lax.fori_loop(0, NSUP // 2, pair_body, 0)

    _gather_wait(0)
    _scatter_wait(0)
    _scatter_wait(1)
    plsc.subcore_barrier()

    for t in range(5):
        row0 = s * WSTRIPE + t * 125
        pltpu.sync_copy(acc.at[pl.ds(row0, 125), pl.ds(0, HALF)], obuf)
        pltpu.sync_copy(obuf, outp_hbm.at[c, pl.ds(row0, 125)])
        pltpu.sync_copy(acc.at[pl.ds(row0, 125), pl.ds(HALF, HALF)], obuf)
        pltpu.sync_copy(obuf, outr_hbm.at[c, pl.ds(row0, 125)])


_BM1 = 2000


def _node_update_body(p_ref, r_ref, w_ref, b_ref, o_ref):
    x = (jnp.dot(p_ref[0], w_ref[:HALF, :], preferred_element_type=jnp.float32)
         + jnp.dot(p_ref[1], w_ref[HALF:, :], preferred_element_type=jnp.float32))
    r = jnp.concatenate([r_ref[0], r_ref[1]], axis=-1)
    h = jnp.maximum(x + b_ref[0] + r, 0.0)
    o_ref[0] = h[:, :HALF]
    o_ref[1] = h[:, HALF:]


_node_update = pl.pallas_call(
    _node_update_body,
    grid=(N_NODES // _BM1,),
    in_specs=[
        pl.BlockSpec((NC, _BM1, HALF), lambda i: (0, i, 0)),
        pl.BlockSpec((NC, _BM1, HALF), lambda i: (0, i, 0)),
        pl.BlockSpec((HIDDEN, HIDDEN), lambda i: (0, 0)),
        pl.BlockSpec((1, HIDDEN), lambda i: (0, 0)),
    ],
    out_specs=pl.BlockSpec((NC, _BM1, HALF), lambda i: (0, i, 0)),
    out_shape=jax.ShapeDtypeStruct((NC, N_NODES, HALF), jnp.float32),
)

_BM2 = 1000


def _final_body(p_ref, r_ref, w_ref, b_ref, wo_ref, bo_ref, o_ref):
    x = (jnp.dot(p_ref[0], w_ref[:HALF, :], preferred_element_type=jnp.float32)
         + jnp.dot(p_ref[1], w_ref[HALF:, :], preferred_element_type=jnp.float32))
    r = jnp.concatenate([r_ref[0], r_ref[1]], axis=-1)
    h = jnp.maximum(x + b_ref[0] + r, 0.0)
    o_ref[...] = jnp.dot(h, wo_ref[...], preferred_element_type=jnp.float32) + bo_ref[0]


_final_mm = pl.pallas_call(
    _final_body,
    grid=(N_NODES // _BM2,),
    in_specs=[
        pl.BlockSpec((NC, _BM2, HALF), lambda i: (0, i, 0)),
        pl.BlockSpec((NC, _BM2, HALF), lambda i: (0, i, 0)),
        pl.BlockSpec((HIDDEN, HIDDEN), lambda i: (0, 0)),
        pl.BlockSpec((1, HIDDEN), lambda i: (0, 0)),
        pl.BlockSpec((HIDDEN, LABELS), lambda i: (0, 0)),
        pl.BlockSpec((1, LABELS), lambda i: (0, 0)),
    ],
    out_specs=pl.BlockSpec((_BM2, LABELS), lambda i: (i, 0)),
    out_shape=jax.ShapeDtypeStruct((N_NODES, LABELS), jnp.float32),
)


def kernel(indices, offsets, edge_index, ppi, self_w, emb_table, input_bias,
           W1, b1, W2, b2, Wout, bout):
    emb_split = emb_table.reshape(INPUT_SIZE, NC, HALF).transpose(1, 0, 2)
    idx_pad = jnp.pad(indices, (0, NNZ_PAD - NNZ))
    offs_pad = jnp.pad(offsets, (0, OFFS_PAD - (N_NODES + 1)),
                       constant_values=NNZ)
    src_pad = jnp.pad(edge_index[0], (0, E_PADX - N_EDGES))
    dst2d = jnp.pad(edge_index[1], (0, E_PADX - N_EDGES),
                    constant_values=TRASH).reshape(E_PADX // CH_G, CH_G)
    ppi_pad = jnp.pad(ppi, (0, E_PADX - N_EDGES))
    sw_pad = jnp.pad(self_w, (0, E_PADX - N_EDGES))

    h = _emb_bag(emb_split, idx_pad, offs_pad, input_bias)
    pp, rr = _gcn_scatter(h, src_pad, dst2d, ppi_pad, sw_pad)
    h = _node_update(pp, rr, W1.T, b1.reshape(1, HIDDEN))
    pp, rr = _gcn_scatter(h, src_pad, dst2d, ppi_pad, sw_pad)
    return _final_mm(pp, rr, W2.T, b2.reshape(1, HIDDEN),
                     Wout.T, bout.reshape(1, LABELS))


# speedup vs baseline: 30.5321x; 1.4303x over previous
import functools

import jax
import jax.numpy as jnp
from jax import lax
from jax.experimental import pallas as pl
from jax.experimental.pallas import tpu as pltpu
from jax.experimental.pallas import tpu_sc as plsc

N_NODES = 10000
N_EDGES = 320000
NNZ = 300000
INPUT_SIZE = 40000
HIDDEN = 128
LABELS = 2000
HALF = 64

NC = 2
NS = 16

CH_E = 128
KI = 148
NNZ_TILE = KI * CH_E
NNZ_PAD = NNZ_TILE * NS
OFFS_PAD = 10016

CH_G = 128
SUP = 8
SUPE = SUP * CH_G
NSUP = 20
E_TILE = NSUP * SUPE
E_PAD = E_TILE * NS
E_PADX = E_PAD + 2 * SUPE

ACC_ROWS = 10240
TRASH = N_NODES
WSTRIPE = N_NODES // NS

_mesh = plsc.VectorSubcoreMesh(core_axis_name="c", subcore_axis_name="s")
_params = pltpu.CompilerParams(use_tc_tiling_on_sc=False,
                               needs_layout_passes=False)


def _zero_rows(buf, n):
    w = 32 if buf.dtype == jnp.bfloat16 else 16
    z = jnp.zeros((w,), buf.dtype)
    cols = buf.shape[1]

    def zrow(r, _):
        for u in range(cols // w):
            buf[r, pl.ds(u * w, w)] = z
        return 0

    lax.fori_loop(0, n, zrow, 0)


@functools.partial(
    pl.kernel,
    out_type=jax.ShapeDtypeStruct((NC, N_NODES, HALF), jnp.float32),
    mesh=_mesh,
    compiler_params=_params,
    scratch_types=[
        pltpu.VMEM((NNZ_TILE,), jnp.int32),
        pltpu.VMEM((4, CH_E), jnp.int32),
        pltpu.VMEM((OFFS_PAD,), jnp.int32),
        [pltpu.VMEM((CH_E, HALF), jnp.float32) for _ in range(4)],
        pltpu.VMEM((125, HALF), jnp.float32),
        pltpu.VMEM((HALF,), jnp.float32),
        pltpu.VMEM_SHARED((ACC_ROWS, HALF), jnp.float32),
        [pltpu.SemaphoreType.DMA for _ in range(4)],
        [pltpu.SemaphoreType.DMA for _ in range(4)],
    ],
)
def _emb_bag(emb_hbm, idx_hbm, offs_hbm, bias_hbm, out_hbm,
             idx_v, seg_v, offs_v, gbufs, obuf, bias_v, acc, gsems, ssems):
    c = lax.axis_index("c")
    s = lax.axis_index("s")

    _zero_rows(gbufs[0], CH_E)
    _zero_rows(gbufs[1], CH_E)
    _zero_rows(gbufs[2], CH_E)
    _zero_rows(gbufs[3], CH_E)
    for t in range(5):
        pltpu.sync_copy(gbufs[t % 4], acc.at[pl.ds(s * 640 + t * CH_E, CH_E)])
    pltpu.sync_copy(offs_hbm, offs_v)
    pltpu.sync_copy(idx_hbm.at[pl.ds(s * NNZ_TILE, NNZ_TILE)], idx_v)
    pltpu.sync_copy(bias_hbm.at[pl.ds(c * HALF, HALF)], bias_v)
    plsc.subcore_barrier()
    for b in range(4):
        pltpu.async_copy(gbufs[b], acc.at[pl.ds(10048, CH_E)], ssems[b])

    pbase = s * NNZ_TILE
    emb_c = emb_hbm.at[c]

    def chunk_group(k4, _):
        for b in range(4):
            j = k4 * 4 + b
            pltpu.make_async_copy(
                gbufs[b], acc.at[pl.ds(0, CH_E)], ssems[b]).wait()
            pltpu.async_copy(emb_c.at[idx_v.at[pl.ds(j * CH_E, CH_E)]],
                             gbufs[b], gsems[b])
            ps = tuple(pbase + j * CH_E + u * 16 + lax.iota(jnp.int32, 16)
                       for u in range(8))
            lo0 = tuple(jnp.zeros((16,), jnp.int32) for _ in range(8))
            hi0 = tuple(jnp.full((16,), 10001, jnp.int32) for _ in range(8))

            def bs(_, lohi):
                los, his = lohi
                nlo, nhi = [], []
                for u in range(8):
                    mid = (los[u] + his[u]) >> 1
                    v = plsc.load_gather(offs_v, [mid])
                    cge = v <= ps[u]
                    nlo.append(jnp.where(cge, mid + 1, los[u]))
                    nhi.append(jnp.where(cge, his[u], mid))
                return (tuple(nlo), tuple(nhi))

            los, his = lax.fori_loop(0, 14, bs, (lo0, hi0))
            for u in range(8):
                seg_v[b, pl.ds(u * 16, 16)] = los[u] - 1
            pltpu.make_async_copy(
                emb_c.at[idx_v.at[pl.ds(j * CH_E, CH_E)]],
                gbufs[b], gsems[b]).wait()
            pltpu.async_copy(gbufs[b], acc.at[seg_v.at[b]], ssems[b],
                             add=True)
        return 0

    lax.fori_loop(0, KI // 4, chunk_group, 0)
    for b in range(4):
        pltpu.make_async_copy(
            gbufs[b], acc.at[pl.ds(0, CH_E)], ssems[b]).wait()
    plsc.subcore_barrier()

    for t in range(5):
        row0 = s * WSTRIPE + t * 125
        pltpu.sync_copy(acc.at[pl.ds(row0, 125)], obuf)

        @plsc.parallel_loop(0, 125, unroll=4)
        def orow(r):
            for u in range(4):
                x = obuf[r, pl.ds(u * 16, 16)] + bias_v[pl.ds(u * 16, 16)]
                obuf[r, pl.ds(u * 16, 16)] = jnp.maximum(x, 0.0)
        pltpu.sync_copy(obuf, out_hbm.at[c, pl.ds(row0, 125)])


@functools.partial(
    pl.kernel,
    out_type=(jax.ShapeDtypeStruct((NC, N_NODES, HALF), jnp.bfloat16),
              jax.ShapeDtypeStruct((NC, N_NODES, HALF), jnp.bfloat16)),
    mesh=_mesh,
    compiler_params=_params,
    scratch_types=[
        pltpu.VMEM((2, SUPE), jnp.int32),
        pltpu.VMEM((2, SUP, CH_G), jnp.int32),
        pltpu.VMEM((2, SUPE), jnp.float32),
        pltpu.VMEM((2, SUPE), jnp.float32),
        [pltpu.VMEM((CH_G, HALF), jnp.bfloat16) for _ in range(2)],
        [pltpu.VMEM((CH_G, HIDDEN), jnp.bfloat16) for _ in range(2)],
        pltpu.VMEM((125, HALF), jnp.bfloat16),
        pltpu.VMEM_SHARED((ACC_ROWS, HIDDEN), jnp.bfloat16),
        [pltpu.SemaphoreType.DMA for _ in range(2)],
        [pltpu.SemaphoreType.DMA for _ in range(2)],
        pltpu.SemaphoreType.DMA,
    ],
)
def _gcn_scatter(h_hbm, src_hbm, dst_hbm, ppi_hbm, sw_hbm, outp_hbm, outr_hbm,
                 src_v, dst_v, ppi_v, sw_v, gbufs, sbufs, obuf, acc,
                 gsems, ssems, msem):
    c = lax.axis_index("c")
    s = lax.axis_index("s")
    h_c = h_hbm.at[c]
    ebase = s * E_TILE
    dbase = s * (E_TILE // CH_G)

    def _meta_fire(g, slot):
        pltpu.async_copy(src_hbm.at[pl.ds(ebase + g * SUPE, SUPE)],
                         src_v.at[slot], msem)
        pltpu.async_copy(dst_hbm.at[pl.ds(dbase + g * SUP, SUP)],
                         dst_v.at[slot], msem)
        pltpu.async_copy(ppi_hbm.at[pl.ds(ebase + g * SUPE, SUPE)],
                         ppi_v.at[slot], msem)
        pltpu.async_copy(sw_hbm.at[pl.ds(ebase + g * SUPE, SUPE)],
                         sw_v.at[slot], msem)

    def _meta_wait(slot):
        pltpu.make_async_copy(src_hbm.at[pl.ds(0, SUPE)],
                              src_v.at[slot], msem).wait()
        pltpu.make_async_copy(dst_hbm.at[pl.ds(0, SUP)],
                              dst_v.at[slot], msem).wait()
        pltpu.make_async_copy(ppi_hbm.at[pl.ds(0, SUPE)],
                              ppi_v.at[slot], msem).wait()
        pltpu.make_async_copy(sw_hbm.at[pl.ds(0, SUPE)],
                              sw_v.at[slot], msem).wait()

    def _gather_fire(slot, jj, b):
        pltpu.async_copy(
            h_c.at[src_v.at[slot].at[pl.ds(jj * CH_G, CH_G)]],
            gbufs[b], gsems[b])

    def _gather_wait(b):
        pltpu.make_async_copy(h_c.at[src_v.at[0].at[pl.ds(0, CH_G)]],
                              gbufs[b], gsems[b]).wait()

    def _scatter_wait(b):
        pltpu.make_async_copy(sbufs[b], acc.at[pl.ds(0, CH_G)],
                              ssems[b]).wait()

    _zero_rows(sbufs[0], CH_G)
    _zero_rows(sbufs[1], CH_G)
    for t in range(5):
        pltpu.sync_copy(sbufs[t % 2],
                        acc.at[pl.ds(s * 640 + t * CH_G, CH_G)])
    plsc.subcore_barrier()
    for b in range(2):
        pltpu.async_copy(sbufs[b], acc.at[pl.ds(10048 + b * 64, CH_G)],
                         ssems[b])

    _meta_fire(0, 0)
    _meta_wait(0)
    _gather_fire(0, 0, 0)

    def pair_body(i, _):
        for m in range(2):
            g = i * 2 + m
            for jj in range(SUP):
                b = jj % 2
                _gather_wait(b)
                if jj < SUP - 1:
                    _gather_fire(m, jj + 1, b ^ 1)
                else:
                    _meta_wait(m ^ 1)
                    _gather_fire(m ^ 1, 0, b ^ 1)
                _scatter_wait(b)
                if jj == 1:
                    _meta_fire(g + 1, m ^ 1)

                @plsc.parallel_loop(0, CH_G, unroll=4)
                def row(r):
                    e = jnp.full((16,), jj * CH_G, jnp.int32) + r
                    sm = jnp.full((16,), m, jnp.int32)
                    wpf = plsc.load_gather(ppi_v, [sm, e])
                    wsf = plsc.load_gather(sw_v, [sm, e])
                    wp = plsc.pack(wpf, wpf, format=plsc.PackFormat.INTERLEAVED)
                    ws = plsc.pack(wsf, wsf, format=plsc.PackFormat.INTERLEAVED)
                    for u in range(2):
                        x = gbufs[b][r, pl.ds(u * 32, 32)]
                        sbufs[b][r, pl.ds(u * 32, 32)] = x * wp
                        sbufs[b][r, pl.ds(HALF + u * 32, 32)] = x * ws
                pltpu.async_copy(sbufs[b], acc.at[dst_v.at[m, jj]],
                                 ssems[b], add=True)
        return 0

    lax.fori_loop(0, NSUP // 2, pair_body, 0)

    _gather_wait(0)
    _scatter_wait(0)
    _scatter_wait(1)
    plsc.subcore_barrier()

    for t in range(5):
        row0 = s * WSTRIPE + t * 125
        pltpu.sync_copy(acc.at[pl.ds(row0, 125), pl.ds(0, HALF)], obuf)
        pltpu.sync_copy(obuf, outp_hbm.at[c, pl.ds(row0, 125)])
        pltpu.sync_copy(acc.at[pl.ds(row0, 125), pl.ds(HALF, HALF)], obuf)
        pltpu.sync_copy(obuf, outr_hbm.at[c, pl.ds(row0, 125)])


_BM1 = 2000


def _node_update_body(p_ref, r_ref, w_ref, b_ref, o_ref):
    x = (jnp.dot(p_ref[0], w_ref[:HALF, :], preferred_element_type=jnp.float32)
         + jnp.dot(p_ref[1], w_ref[HALF:, :], preferred_element_type=jnp.float32))
    r = jnp.concatenate([r_ref[0], r_ref[1]], axis=-1)
    h = jnp.maximum(x + b_ref[0] + r, 0.0)
    o_ref[0] = h[:, :HALF]
    o_ref[1] = h[:, HALF:]


_node_update = pl.pallas_call(
    _node_update_body,
    grid=(N_NODES // _BM1,),
    in_specs=[
        pl.BlockSpec((NC, _BM1, HALF), lambda i: (0, i, 0)),
        pl.BlockSpec((NC, _BM1, HALF), lambda i: (0, i, 0)),
        pl.BlockSpec((HIDDEN, HIDDEN), lambda i: (0, 0)),
        pl.BlockSpec((1, HIDDEN), lambda i: (0, 0)),
    ],
    out_specs=pl.BlockSpec((NC, _BM1, HALF), lambda i: (0, i, 0)),
    out_shape=jax.ShapeDtypeStruct((NC, N_NODES, HALF), jnp.float32),
)

_BM2 = 1000


def _final_body(p_ref, r_ref, w_ref, b_ref, wo_ref, bo_ref, o_ref):
    x = (jnp.dot(p_ref[0], w_ref[:HALF, :], preferred_element_type=jnp.float32)
         + jnp.dot(p_ref[1], w_ref[HALF:, :], preferred_element_type=jnp.float32))
    r = jnp.concatenate([r_ref[0], r_ref[1]], axis=-1)
    h = jnp.maximum(x + b_ref[0] + r, 0.0)
    o_ref[...] = jnp.dot(h, wo_ref[...], preferred_element_type=jnp.float32) + bo_ref[0]


_final_mm = pl.pallas_call(
    _final_body,
    grid=(N_NODES // _BM2,),
    in_specs=[
        pl.BlockSpec((NC, _BM2, HALF), lambda i: (0, i, 0)),
        pl.BlockSpec((NC, _BM2, HALF), lambda i: (0, i, 0)),
        pl.BlockSpec((HIDDEN, HIDDEN), lambda i: (0, 0)),
        pl.BlockSpec((1, HIDDEN), lambda i: (0, 0)),
        pl.BlockSpec((HIDDEN, LABELS), lambda i: (0, 0)),
        pl.BlockSpec((1, LABELS), lambda i: (0, 0)),
    ],
    out_specs=pl.BlockSpec((_BM2, LABELS), lambda i: (i, 0)),
    out_shape=jax.ShapeDtypeStruct((N_NODES, LABELS), jnp.float32),
)


def kernel(indices, offsets, edge_index, ppi, self_w, emb_table, input_bias,
           W1, b1, W2, b2, Wout, bout):
    emb_split = emb_table.reshape(INPUT_SIZE, NC, HALF).transpose(1, 0, 2)
    idx_pad = jnp.pad(indices, (0, NNZ_PAD - NNZ))
    offs_pad = jnp.pad(offsets, (0, OFFS_PAD - (N_NODES + 1)),
                       constant_values=NNZ)
    src_pad = jnp.pad(edge_index[0], (0, E_PADX - N_EDGES))
    dst2d = jnp.pad(edge_index[1], (0, E_PADX - N_EDGES),
                    constant_values=TRASH).reshape(E_PADX // CH_G, CH_G)
    ppi_pad = jnp.pad(ppi, (0, E_PADX - N_EDGES))
    sw_pad = jnp.pad(self_w, (0, E_PADX - N_EDGES))

    h = _emb_bag(emb_split, idx_pad, offs_pad, input_bias)
    pp, rr = _gcn_scatter(h.astype(jnp.bfloat16), src_pad, dst2d,
                          ppi_pad, sw_pad)
    h = _node_update(pp, rr, W1.T, b1.reshape(1, HIDDEN))
    pp, rr = _gcn_scatter(h.astype(jnp.bfloat16), src_pad, dst2d,
                          ppi_pad, sw_pad)
    return _final_mm(pp, rr, W2.T, b2.reshape(1, HIDDEN),
                     Wout.T, bout.reshape(1, LABELS))


# speedup vs baseline: 31.0732x; 1.0177x over previous
import functools

import jax
import jax.numpy as jnp
from jax import lax
from jax.experimental import pallas as pl
from jax.experimental.pallas import tpu as pltpu
from jax.experimental.pallas import tpu_sc as plsc

N_NODES = 10000
N_EDGES = 320000
NNZ = 300000
INPUT_SIZE = 40000
HIDDEN = 128
LABELS = 2000
HALF = 64

NC = 2
NS = 16

CH_E = 128
KI = 148
NNZ_TILE = KI * CH_E
NNZ_PAD = NNZ_TILE * NS
OFFS_PAD = 10016

CH_G = 128
SUP = 8
SUPE = SUP * CH_G
NSUP = 20
E_TILE = NSUP * SUPE
E_PAD = E_TILE * NS
E_PADX = E_PAD + 2 * SUPE

ACC_ROWS = 10240
TRASH = N_NODES
WSTRIPE = N_NODES // NS

_mesh = plsc.VectorSubcoreMesh(core_axis_name="c", subcore_axis_name="s")
_params = pltpu.CompilerParams(use_tc_tiling_on_sc=False,
                               needs_layout_passes=False)


def _zero_rows(buf, n):
    w = 32 if buf.dtype == jnp.bfloat16 else 16
    z = jnp.zeros((w,), buf.dtype)
    cols = buf.shape[1]

    def zrow(r, _):
        for u in range(cols // w):
            buf[r, pl.ds(u * w, w)] = z
        return 0

    lax.fori_loop(0, n, zrow, 0)


@functools.partial(
    pl.kernel,
    out_type=jax.ShapeDtypeStruct((NC, N_NODES, HALF), jnp.bfloat16),
    mesh=_mesh,
    compiler_params=_params,
    scratch_types=[
        pltpu.VMEM((NNZ_TILE,), jnp.int32),
        pltpu.VMEM((4, CH_E), jnp.int32),
        pltpu.VMEM((OFFS_PAD,), jnp.int32),
        [pltpu.VMEM((CH_E, HALF), jnp.float32) for _ in range(4)],
        pltpu.VMEM((125, HALF), jnp.float32),
        pltpu.VMEM((125, HALF), jnp.bfloat16),
        pltpu.VMEM((HALF,), jnp.float32),
        pltpu.VMEM_SHARED((ACC_ROWS, HALF), jnp.float32),
        [pltpu.SemaphoreType.DMA for _ in range(4)],
        [pltpu.SemaphoreType.DMA for _ in range(4)],
    ],
)
def _emb_bag(emb_hbm, idx_hbm, offs_hbm, bias_hbm, out_hbm,
             idx_v, seg_v, offs_v, gbufs, obuf, obuf16, bias_v, acc,
             gsems, ssems):
    c = lax.axis_index("c")
    s = lax.axis_index("s")

    _zero_rows(gbufs[0], CH_E)
    _zero_rows(gbufs[1], CH_E)
    _zero_rows(gbufs[2], CH_E)
    _zero_rows(gbufs[3], CH_E)
    for t in range(5):
        pltpu.sync_copy(gbufs[t % 4], acc.at[pl.ds(s * 640 + t * CH_E, CH_E)])
    pltpu.sync_copy(offs_hbm, offs_v)
    pltpu.sync_copy(idx_hbm.at[pl.ds(s * NNZ_TILE, NNZ_TILE)], idx_v)
    pltpu.sync_copy(bias_hbm.at[pl.ds(c * HALF, HALF)], bias_v)
    plsc.subcore_barrier()
    for b in range(4):
        pltpu.async_copy(gbufs[b], acc.at[pl.ds(10048, CH_E)], ssems[b])

    pbase = s * NNZ_TILE
    emb_c = emb_hbm.at[c]

    def chunk_group(k4, _):
        for b in range(4):
            j = k4 * 4 + b
            pltpu.make_async_copy(
                gbufs[b], acc.at[pl.ds(0, CH_E)], ssems[b]).wait()
            pltpu.async_copy(emb_c.at[idx_v.at[pl.ds(j * CH_E, CH_E)]],
                             gbufs[b], gsems[b])
            ps = tuple(pbase + j * CH_E + u * 16 + lax.iota(jnp.int32, 16)
                       for u in range(8))
            lo0 = tuple(jnp.zeros((16,), jnp.int32) for _ in range(8))
            hi0 = tuple(jnp.full((16,), 10001, jnp.int32) for _ in range(8))

            def bs(_, lohi):
                los, his = lohi
                nlo, nhi = [], []
                for u in range(8):
                    mid = (los[u] + his[u]) >> 1
                    v = plsc.load_gather(offs_v, [mid])
                    cge = v <= ps[u]
                    nlo.append(jnp.where(cge, mid + 1, los[u]))
                    nhi.append(jnp.where(cge, his[u], mid))
                return (tuple(nlo), tuple(nhi))

            los, his = lax.fori_loop(0, 14, bs, (lo0, hi0))
            for u in range(8):
                seg_v[b, pl.ds(u * 16, 16)] = los[u] - 1
            pltpu.make_async_copy(
                emb_c.at[idx_v.at[pl.ds(j * CH_E, CH_E)]],
                gbufs[b], gsems[b]).wait()
            pltpu.async_copy(gbufs[b], acc.at[seg_v.at[b]], ssems[b],
                             add=True)
        return 0

    lax.fori_loop(0, KI // 4, chunk_group, 0)
    for b in range(4):
        pltpu.make_async_copy(
            gbufs[b], acc.at[pl.ds(0, CH_E)], ssems[b]).wait()
    plsc.subcore_barrier()

    ev = lax.iota(jnp.int32, 16) * 2
    for t in range(5):
        row0 = s * WSTRIPE + t * 125
        pltpu.sync_copy(acc.at[pl.ds(row0, 125)], obuf)

        @plsc.parallel_loop(0, 125, unroll=4)
        def orow(r):
            for u in range(4):
                x = obuf[r, pl.ds(u * 16, 16)] + bias_v[pl.ds(u * 16, 16)]
                obuf[r, pl.ds(u * 16, 16)] = jnp.maximum(x, 0.0)
            rr = jnp.full((16,), r, jnp.int32)
            for u in range(2):
                xa = plsc.load_gather(obuf, [rr, ev + u * 32])
                xb = plsc.load_gather(obuf, [rr, ev + u * 32 + 1])
                obuf16[r, pl.ds(u * 32, 32)] = plsc.pack(
                    xa, xb, format=plsc.PackFormat.INTERLEAVED)
        pltpu.sync_copy(obuf16, out_hbm.at[c, pl.ds(row0, 125)])


@functools.partial(
    pl.kernel,
    out_type=(jax.ShapeDtypeStruct((NC, N_NODES, HALF), jnp.bfloat16),
              jax.ShapeDtypeStruct((NC, N_NODES, HALF), jnp.bfloat16)),
    mesh=_mesh,
    compiler_params=_params,
    scratch_types=[
        pltpu.VMEM((2, SUPE), jnp.int32),
        pltpu.VMEM((2, SUP, CH_G), jnp.int32),
        pltpu.VMEM((2, SUPE), jnp.float32),
        pltpu.VMEM((2, SUPE), jnp.float32),
        [pltpu.VMEM((CH_G, HALF), jnp.bfloat16) for _ in range(2)],
        [pltpu.VMEM((CH_G, HIDDEN), jnp.bfloat16) for _ in range(2)],
        pltpu.VMEM((125, HALF), jnp.bfloat16),
        pltpu.VMEM_SHARED((ACC_ROWS, HIDDEN), jnp.bfloat16),
        [pltpu.SemaphoreType.DMA for _ in range(2)],
        [pltpu.SemaphoreType.DMA for _ in range(2)],
        pltpu.SemaphoreType.DMA,
    ],
)
def _gcn_scatter(h_hbm, src_hbm, dst_hbm, ppi_hbm, sw_hbm, outp_hbm, outr_hbm,
                 src_v, dst_v, ppi_v, sw_v, gbufs, sbufs, obuf, acc,
                 gsems, ssems, msem):
    c = lax.axis_index("c")
    s = lax.axis_index("s")
    h_c = h_hbm.at[c]
    ebase = s * E_TILE
    dbase = s * (E_TILE // CH_G)

    def _meta_fire(g, slot):
        pltpu.async_copy(src_hbm.at[pl.ds(ebase + g * SUPE, SUPE)],
                         src_v.at[slot], msem)
        pltpu.async_copy(dst_hbm.at[pl.ds(dbase + g * SUP, SUP)],
                         dst_v.at[slot], msem)
        pltpu.async_copy(ppi_hbm.at[pl.ds(ebase + g * SUPE, SUPE)],
                         ppi_v.at[slot], msem)
        pltpu.async_copy(sw_hbm.at[pl.ds(ebase + g * SUPE, SUPE)],
                         sw_v.at[slot], msem)

    def _meta_wait(slot):
        pltpu.make_async_copy(src_hbm.at[pl.ds(0, SUPE)],
                              src_v.at[slot], msem).wait()
        pltpu.make_async_copy(dst_hbm.at[pl.ds(0, SUP)],
                              dst_v.at[slot], msem).wait()
        pltpu.make_async_copy(ppi_hbm.at[pl.ds(0, SUPE)],
                              ppi_v.at[slot], msem).wait()
        pltpu.make_async_copy(sw_hbm.at[pl.ds(0, SUPE)],
                              sw_v.at[slot], msem).wait()

    def _gather_fire(slot, jj, b):
        pltpu.async_copy(
            h_c.at[src_v.at[slot].at[pl.ds(jj * CH_G, CH_G)]],
            gbufs[b], gsems[b])

    def _gather_wait(b):
        pltpu.make_async_copy(h_c.at[src_v.at[0].at[pl.ds(0, CH_G)]],
                              gbufs[b], gsems[b]).wait()

    def _scatter_wait(b):
        pltpu.make_async_copy(sbufs[b], acc.at[pl.ds(0, CH_G)],
                              ssems[b]).wait()

    _zero_rows(sbufs[0], CH_G)
    _zero_rows(sbufs[1], CH_G)
    for t in range(5):
        pltpu.sync_copy(sbufs[t % 2],
                        acc.at[pl.ds(s * 640 + t * CH_G, CH_G)])
    plsc.subcore_barrier()
    for b in range(2):
        pltpu.async_copy(sbufs[b], acc.at[pl.ds(10048 + b * 64, CH_G)],
                         ssems[b])

    _meta_fire(0, 0)
    _meta_wait(0)
    _gather_fire(0, 0, 0)

    def pair_body(i, _):
        for m in range(2):
            g = i * 2 + m
            for jj in range(SUP):
                b = jj % 2
                _gather_wait(b)
                if jj < SUP - 1:
                    _gather_fire(m, jj + 1, b ^ 1)
                else:
                    _meta_wait(m ^ 1)
                    _gather_fire(m ^ 1, 0, b ^ 1)
                _scatter_wait(b)
                if jj == 1:
                    _meta_fire(g + 1, m ^ 1)

                @plsc.parallel_loop(0, CH_G, unroll=4)
                def row(r):
                    e = jnp.full((16,), jj * CH_G, jnp.int32) + r
                    sm = jnp.full((16,), m, jnp.int32)
                    wpf = plsc.load_gather(ppi_v, [sm, e])
                    wsf = plsc.load_gather(sw_v, [sm, e])
                    wp = plsc.pack(wpf, wpf, format=plsc.PackFormat.INTERLEAVED)
                    ws = plsc.pack(wsf, wsf, format=plsc.PackFormat.INTERLEAVED)
                    for u in range(2):
                        x = gbufs[b][r, pl.ds(u * 32, 32)]
                        sbufs[b][r, pl.ds(u * 32, 32)] = x * wp
                        sbufs[b][r, pl.ds(HALF + u * 32, 32)] = x * ws
                pltpu.async_copy(sbufs[b], acc.at[dst_v.at[m, jj]],
                                 ssems[b], add=True)
        return 0

    lax.fori_loop(0, NSUP // 2, pair_body, 0)

    _gather_wait(0)
    _scatter_wait(0)
    _scatter_wait(1)
    plsc.subcore_barrier()

    for t in range(5):
        row0 = s * WSTRIPE + t * 125
        pltpu.sync_copy(acc.at[pl.ds(row0, 125), pl.ds(0, HALF)], obuf)
        pltpu.sync_copy(obuf, outp_hbm.at[c, pl.ds(row0, 125)])
        pltpu.sync_copy(acc.at[pl.ds(row0, 125), pl.ds(HALF, HALF)], obuf)
        pltpu.sync_copy(obuf, outr_hbm.at[c, pl.ds(row0, 125)])


_BM1 = 2000


def _node_update_body(p_ref, r_ref, w_ref, b_ref, o_ref):
    x = (jnp.dot(p_ref[0], w_ref[:HALF, :], preferred_element_type=jnp.float32)
         + jnp.dot(p_ref[1], w_ref[HALF:, :], preferred_element_type=jnp.float32))
    r = jnp.concatenate([r_ref[0], r_ref[1]], axis=-1)
    h = jnp.maximum(x + b_ref[0] + r, 0.0).astype(jnp.bfloat16)
    o_ref[0] = h[:, :HALF]
    o_ref[1] = h[:, HALF:]


_node_update = pl.pallas_call(
    _node_update_body,
    grid=(N_NODES // _BM1,),
    in_specs=[
        pl.BlockSpec((NC, _BM1, HALF), lambda i: (0, i, 0)),
        pl.BlockSpec((NC, _BM1, HALF), lambda i: (0, i, 0)),
        pl.BlockSpec((HIDDEN, HIDDEN), lambda i: (0, 0)),
        pl.BlockSpec((1, HIDDEN), lambda i: (0, 0)),
    ],
    out_specs=pl.BlockSpec((NC, _BM1, HALF), lambda i: (0, i, 0)),
    out_shape=jax.ShapeDtypeStruct((NC, N_NODES, HALF), jnp.bfloat16),
)

_BM2 = 1000


def _final_body(p_ref, r_ref, w_ref, b_ref, wo_ref, bo_ref, o_ref):
    x = (jnp.dot(p_ref[0], w_ref[:HALF, :], preferred_element_type=jnp.float32)
         + jnp.dot(p_ref[1], w_ref[HALF:, :], preferred_element_type=jnp.float32))
    r = jnp.concatenate([r_ref[0], r_ref[1]], axis=-1)
    h = jnp.maximum(x + b_ref[0] + r, 0.0)
    o_ref[...] = jnp.dot(h, wo_ref[...], preferred_element_type=jnp.float32) + bo_ref[0]


_final_mm = pl.pallas_call(
    _final_body,
    grid=(N_NODES // _BM2,),
    in_specs=[
        pl.BlockSpec((NC, _BM2, HALF), lambda i: (0, i, 0)),
        pl.BlockSpec((NC, _BM2, HALF), lambda i: (0, i, 0)),
        pl.BlockSpec((HIDDEN, HIDDEN), lambda i: (0, 0)),
        pl.BlockSpec((1, HIDDEN), lambda i: (0, 0)),
        pl.BlockSpec((HIDDEN, LABELS), lambda i: (0, 0)),
        pl.BlockSpec((1, LABELS), lambda i: (0, 0)),
    ],
    out_specs=pl.BlockSpec((_BM2, LABELS), lambda i: (i, 0)),
    out_shape=jax.ShapeDtypeStruct((N_NODES, LABELS), jnp.float32),
)


def kernel(indices, offsets, edge_index, ppi, self_w, emb_table, input_bias,
           W1, b1, W2, b2, Wout, bout):
    emb_split = emb_table.reshape(INPUT_SIZE, NC, HALF).transpose(1, 0, 2)
    idx_pad = jnp.pad(indices, (0, NNZ_PAD - NNZ))
    offs_pad = jnp.pad(offsets, (0, OFFS_PAD - (N_NODES + 1)),
                       constant_values=NNZ)
    src_pad = jnp.pad(edge_index[0], (0, E_PADX - N_EDGES))
    dst2d = jnp.pad(edge_index[1], (0, E_PADX - N_EDGES),
                    constant_values=TRASH).reshape(E_PADX // CH_G, CH_G)
    ppi_pad = jnp.pad(ppi, (0, E_PADX - N_EDGES))
    sw_pad = jnp.pad(self_w, (0, E_PADX - N_EDGES))

    h = _emb_bag(emb_split, idx_pad, offs_pad, input_bias)
    pp, rr = _gcn_scatter(h, src_pad, dst2d, ppi_pad, sw_pad)
    h = _node_update(pp, rr, W1.T, b1.reshape(1, HIDDEN))
    pp, rr = _gcn_scatter(h, src_pad, dst2d, ppi_pad, sw_pad)
    return _final_mm(pp, rr, W2.T, b2.reshape(1, HIDDEN),
                     Wout.T, bout.reshape(1, LABELS))


# speedup vs baseline: 32.8630x; 1.0576x over previous
import functools

import jax
import jax.numpy as jnp
from jax import lax
from jax.experimental import pallas as pl
from jax.experimental.pallas import tpu as pltpu
from jax.experimental.pallas import tpu_sc as plsc

N_NODES = 10000
N_EDGES = 320000
NNZ = 300000
INPUT_SIZE = 40000
HIDDEN = 128
LABELS = 2000
HALF = 64

NC = 2
NS = 16

CH_E = 128
KI = 148
NNZ_TILE = KI * CH_E
NNZ_PAD = NNZ_TILE * NS
OFFS_PAD = 10016

CH_G = 128
SUP = 8
SUPE = SUP * CH_G
NSUP = 20
E_TILE = NSUP * SUPE
E_PAD = E_TILE * NS
E_PADX = E_PAD + 2 * SUPE

ACC_ROWS = 10240
TRASH = N_NODES
WSTRIPE = N_NODES // NS

_mesh = plsc.VectorSubcoreMesh(core_axis_name="c", subcore_axis_name="s")
_params = pltpu.CompilerParams(use_tc_tiling_on_sc=False,
                               needs_layout_passes=False)


def _zero_rows(buf, n):
    w = 32 if buf.dtype == jnp.bfloat16 else 16
    z = jnp.zeros((w,), buf.dtype)
    cols = buf.shape[1]

    def zrow(r, _):
        for u in range(cols // w):
            buf[r, pl.ds(u * w, w)] = z
        return 0

    lax.fori_loop(0, n, zrow, 0)


@functools.partial(
    pl.kernel,
    out_type=jax.ShapeDtypeStruct((NC, N_NODES, HALF), jnp.bfloat16),
    mesh=_mesh,
    compiler_params=_params,
    scratch_types=[
        pltpu.VMEM((NNZ_TILE,), jnp.int32),
        pltpu.VMEM((4, CH_E), jnp.int32),
        pltpu.VMEM((OFFS_PAD,), jnp.int32),
        [pltpu.VMEM((CH_E, HALF), jnp.float32) for _ in range(4)],
        pltpu.VMEM((125, HALF), jnp.float32),
        pltpu.VMEM((125, HALF), jnp.bfloat16),
        pltpu.VMEM((HALF,), jnp.float32),
        pltpu.VMEM_SHARED((ACC_ROWS, HALF), jnp.float32),
        [pltpu.SemaphoreType.DMA for _ in range(4)],
        [pltpu.SemaphoreType.DMA for _ in range(4)],
    ],
)
def _emb_bag(emb_hbm, idx_hbm, offs_hbm, bias_hbm, out_hbm,
             idx_v, seg_v, offs_v, gbufs, obuf, obuf16, bias_v, acc,
             gsems, ssems):
    c = lax.axis_index("c")
    s = lax.axis_index("s")

    _zero_rows(gbufs[0], CH_E)
    _zero_rows(gbufs[1], CH_E)
    _zero_rows(gbufs[2], CH_E)
    _zero_rows(gbufs[3], CH_E)
    for t in range(5):
        pltpu.sync_copy(gbufs[t % 4], acc.at[pl.ds(s * 640 + t * CH_E, CH_E)])
    pltpu.sync_copy(offs_hbm, offs_v)
    pltpu.sync_copy(idx_hbm.at[pl.ds(s * NNZ_TILE, NNZ_TILE)], idx_v)
    pltpu.sync_copy(bias_hbm.at[pl.ds(c * HALF, HALF)], bias_v)

    @plsc.parallel_loop(0, NNZ_TILE // 16, unroll=8)
    def ixf(k):
        v = idx_v[pl.ds(k * 16, 16)]
        idx_v[pl.ds(k * 16, 16)] = v * 2 + c

    plsc.subcore_barrier()
    for b in range(4):
        pltpu.async_copy(gbufs[b], acc.at[pl.ds(10048, CH_E)], ssems[b])

    pbase = s * NNZ_TILE
    emb_c = emb_hbm

    def chunk_group(k4, _):
        for b in range(4):
            j = k4 * 4 + b
            pltpu.make_async_copy(
                gbufs[b], acc.at[pl.ds(0, CH_E)], ssems[b]).wait()
            pltpu.async_copy(emb_c.at[idx_v.at[pl.ds(j * CH_E, CH_E)]],
                             gbufs[b], gsems[b])
            ps = tuple(pbase + j * CH_E + u * 16 + lax.iota(jnp.int32, 16)
                       for u in range(8))
            lo0 = tuple(jnp.zeros((16,), jnp.int32) for _ in range(8))
            hi0 = tuple(jnp.full((16,), 10001, jnp.int32) for _ in range(8))

            def bs(_, lohi):
                los, his = lohi
                nlo, nhi = [], []
                for u in range(8):
                    mid = (los[u] + his[u]) >> 1
                    v = plsc.load_gather(offs_v, [mid])
                    cge = v <= ps[u]
                    nlo.append(jnp.where(cge, mid + 1, los[u]))
                    nhi.append(jnp.where(cge, his[u], mid))
                return (tuple(nlo), tuple(nhi))

            los, his = lax.fori_loop(0, 14, bs, (lo0, hi0))
            for u in range(8):
                seg_v[b, pl.ds(u * 16, 16)] = los[u] - 1
            pltpu.make_async_copy(
                emb_c.at[idx_v.at[pl.ds(j * CH_E, CH_E)]],
                gbufs[b], gsems[b]).wait()
            pltpu.async_copy(gbufs[b], acc.at[seg_v.at[b]], ssems[b],
                             add=True)
        return 0

    lax.fori_loop(0, KI // 4, chunk_group, 0)
    for b in range(4):
        pltpu.make_async_copy(
            gbufs[b], acc.at[pl.ds(0, CH_E)], ssems[b]).wait()
    plsc.subcore_barrier()

    ev = lax.iota(jnp.int32, 16) * 2
    for t in range(5):
        row0 = s * WSTRIPE + t * 125
        pltpu.sync_copy(acc.at[pl.ds(row0, 125)], obuf)

        @plsc.parallel_loop(0, 125, unroll=4)
        def orow(r):
            for u in range(4):
                x = obuf[r, pl.ds(u * 16, 16)] + bias_v[pl.ds(u * 16, 16)]
                obuf[r, pl.ds(u * 16, 16)] = jnp.maximum(x, 0.0)
            rr = jnp.full((16,), r, jnp.int32)
            for u in range(2):
                xa = plsc.load_gather(obuf, [rr, ev + u * 32])
                xb = plsc.load_gather(obuf, [rr, ev + u * 32 + 1])
                obuf16[r, pl.ds(u * 32, 32)] = plsc.pack(
                    xa, xb, format=plsc.PackFormat.INTERLEAVED)
        pltpu.sync_copy(obuf16, out_hbm.at[c, pl.ds(row0, 125)])


@functools.partial(
    pl.kernel,
    out_type=(jax.ShapeDtypeStruct((NC, N_NODES, HALF), jnp.bfloat16),
              jax.ShapeDtypeStruct((NC, N_NODES, HALF), jnp.bfloat16)),
    mesh=_mesh,
    compiler_params=_params,
    scratch_types=[
        pltpu.VMEM((2, SUPE), jnp.int32),
        pltpu.VMEM((2, SUP, CH_G), jnp.int32),
        pltpu.VMEM((2, SUPE), jnp.float32),
        pltpu.VMEM((2, SUPE), jnp.float32),
        [pltpu.VMEM((CH_G, HALF), jnp.bfloat16) for _ in range(2)],
        [pltpu.VMEM((CH_G, HIDDEN), jnp.bfloat16) for _ in range(2)],
        pltpu.VMEM((125, HALF), jnp.bfloat16),
        pltpu.VMEM_SHARED((ACC_ROWS, HIDDEN), jnp.bfloat16),
        [pltpu.SemaphoreType.DMA for _ in range(2)],
        [pltpu.SemaphoreType.DMA for _ in range(2)],
        pltpu.SemaphoreType.DMA,
    ],
)
def _gcn_scatter(h_hbm, src_hbm, dst_hbm, ppi_hbm, sw_hbm, outp_hbm, outr_hbm,
                 src_v, dst_v, ppi_v, sw_v, gbufs, sbufs, obuf, acc,
                 gsems, ssems, msem):
    c = lax.axis_index("c")
    s = lax.axis_index("s")
    h_c = h_hbm.at[c]
    ebase = s * E_TILE
    dbase = s * (E_TILE // CH_G)

    def _meta_fire(g, slot):
        pltpu.async_copy(src_hbm.at[pl.ds(ebase + g * SUPE, SUPE)],
                         src_v.at[slot], msem)
        pltpu.async_copy(dst_hbm.at[pl.ds(dbase + g * SUP, SUP)],
                         dst_v.at[slot], msem)
        pltpu.async_copy(ppi_hbm.at[pl.ds(ebase + g * SUPE, SUPE)],
                         ppi_v.at[slot], msem)
        pltpu.async_copy(sw_hbm.at[pl.ds(ebase + g * SUPE, SUPE)],
                         sw_v.at[slot], msem)

    def _meta_wait(slot):
        pltpu.make_async_copy(src_hbm.at[pl.ds(0, SUPE)],
                              src_v.at[slot], msem).wait()
        pltpu.make_async_copy(dst_hbm.at[pl.ds(0, SUP)],
                              dst_v.at[slot], msem).wait()
        pltpu.make_async_copy(ppi_hbm.at[pl.ds(0, SUPE)],
                              ppi_v.at[slot], msem).wait()
        pltpu.make_async_copy(sw_hbm.at[pl.ds(0, SUPE)],
                              sw_v.at[slot], msem).wait()

    def _gather_fire(slot, jj, b):
        pltpu.async_copy(
            h_c.at[src_v.at[slot].at[pl.ds(jj * CH_G, CH_G)]],
            gbufs[b], gsems[b])

    def _gather_wait(b):
        pltpu.make_async_copy(h_c.at[src_v.at[0].at[pl.ds(0, CH_G)]],
                              gbufs[b], gsems[b]).wait()

    def _scatter_wait(b):
        pltpu.make_async_copy(sbufs[b], acc.at[pl.ds(0, CH_G)],
                              ssems[b]).wait()

    _zero_rows(sbufs[0], CH_G)
    _zero_rows(sbufs[1], CH_G)
    for t in range(5):
        pltpu.sync_copy(sbufs[t % 2],
                        acc.at[pl.ds(s * 640 + t * CH_G, CH_G)])
    plsc.subcore_barrier()
    for b in range(2):
        pltpu.async_copy(sbufs[b], acc.at[pl.ds(10048 + b * 64, CH_G)],
                         ssems[b])

    _meta_fire(0, 0)
    _meta_wait(0)
    _gather_fire(0, 0, 0)

    def pair_body(i, _):
        for m in range(2):
            g = i * 2 + m
            for jj in range(SUP):
                b = jj % 2
                _gather_wait(b)
                if jj < SUP - 1:
                    _gather_fire(m, jj + 1, b ^ 1)
                else:
                    _meta_wait(m ^ 1)
                    _gather_fire(m ^ 1, 0, b ^ 1)
                _scatter_wait(b)
                if jj == 1:
                    _meta_fire(g + 1, m ^ 1)

                @plsc.parallel_loop(0, CH_G, unroll=4)
                def row(r):
                    e = jnp.full((16,), jj * CH_G, jnp.int32) + r
                    sm = jnp.full((16,), m, jnp.int32)
                    wpf = plsc.load_gather(ppi_v, [sm, e])
                    wsf = plsc.load_gather(sw_v, [sm, e])
                    wp = plsc.pack(wpf, wpf, format=plsc.PackFormat.INTERLEAVED)
                    ws = plsc.pack(wsf, wsf, format=plsc.PackFormat.INTERLEAVED)
                    for u in range(2):
                        x = gbufs[b][r, pl.ds(u * 32, 32)]
                        sbufs[b][r, pl.ds(u * 32, 32)] = x * wp
                        sbufs[b][r, pl.ds(HALF + u * 32, 32)] = x * ws
                pltpu.async_copy(sbufs[b], acc.at[dst_v.at[m, jj]],
                                 ssems[b], add=True)
        return 0

    lax.fori_loop(0, NSUP // 2, pair_body, 0)

    _gather_wait(0)
    _scatter_wait(0)
    _scatter_wait(1)
    plsc.subcore_barrier()

    for t in range(5):
        row0 = s * WSTRIPE + t * 125
        pltpu.sync_copy(acc.at[pl.ds(row0, 125), pl.ds(0, HALF)], obuf)
        pltpu.sync_copy(obuf, outp_hbm.at[c, pl.ds(row0, 125)])
        pltpu.sync_copy(acc.at[pl.ds(row0, 125), pl.ds(HALF, HALF)], obuf)
        pltpu.sync_copy(obuf, outr_hbm.at[c, pl.ds(row0, 125)])


_BM1 = 2000


def _node_update_body(p_ref, r_ref, w_ref, b_ref, o_ref):
    x = (jnp.dot(p_ref[0], w_ref[:HALF, :], preferred_element_type=jnp.float32)
         + jnp.dot(p_ref[1], w_ref[HALF:, :], preferred_element_type=jnp.float32))
    r = jnp.concatenate([r_ref[0], r_ref[1]], axis=-1)
    h = jnp.maximum(x + b_ref[0] + r, 0.0).astype(jnp.bfloat16)
    o_ref[0] = h[:, :HALF]
    o_ref[1] = h[:, HALF:]


_node_update = pl.pallas_call(
    _node_update_body,
    grid=(N_NODES // _BM1,),
    in_specs=[
        pl.BlockSpec((NC, _BM1, HALF), lambda i: (0, i, 0)),
        pl.BlockSpec((NC, _BM1, HALF), lambda i: (0, i, 0)),
        pl.BlockSpec((HIDDEN, HIDDEN), lambda i: (0, 0)),
        pl.BlockSpec((1, HIDDEN), lambda i: (0, 0)),
    ],
    out_specs=pl.BlockSpec((NC, _BM1, HALF), lambda i: (0, i, 0)),
    out_shape=jax.ShapeDtypeStruct((NC, N_NODES, HALF), jnp.bfloat16),
)

_BM2 = 1000


def _final_body(p_ref, r_ref, w_ref, b_ref, wo_ref, bo_ref, o_ref):
    x = (jnp.dot(p_ref[0], w_ref[:HALF, :], preferred_element_type=jnp.float32)
         + jnp.dot(p_ref[1], w_ref[HALF:, :], preferred_element_type=jnp.float32))
    r = jnp.concatenate([r_ref[0], r_ref[1]], axis=-1)
    h = jnp.maximum(x + b_ref[0] + r, 0.0)
    o_ref[...] = jnp.dot(h, wo_ref[...], preferred_element_type=jnp.float32) + bo_ref[0]


_final_mm = pl.pallas_call(
    _final_body,
    grid=(N_NODES // _BM2,),
    in_specs=[
        pl.BlockSpec((NC, _BM2, HALF), lambda i: (0, i, 0)),
        pl.BlockSpec((NC, _BM2, HALF), lambda i: (0, i, 0)),
        pl.BlockSpec((HIDDEN, HIDDEN), lambda i: (0, 0)),
        pl.BlockSpec((1, HIDDEN), lambda i: (0, 0)),
        pl.BlockSpec((HIDDEN, LABELS), lambda i: (0, 0)),
        pl.BlockSpec((1, LABELS), lambda i: (0, 0)),
    ],
    out_specs=pl.BlockSpec((_BM2, LABELS), lambda i: (i, 0)),
    out_shape=jax.ShapeDtypeStruct((N_NODES, LABELS), jnp.float32),
)


def kernel(indices, offsets, edge_index, ppi, self_w, emb_table, input_bias,
           W1, b1, W2, b2, Wout, bout):
    emb_split = emb_table.reshape(INPUT_SIZE * NC, HALF)
    idx_pad = jnp.pad(indices, (0, NNZ_PAD - NNZ))
    offs_pad = jnp.pad(offsets, (0, OFFS_PAD - (N_NODES + 1)),
                       constant_values=NNZ)
    src_pad = jnp.pad(edge_index[0], (0, E_PADX - N_EDGES))
    dst2d = jnp.pad(edge_index[1], (0, E_PADX - N_EDGES),
                    constant_values=TRASH).reshape(E_PADX // CH_G, CH_G)
    ppi_pad = jnp.pad(ppi, (0, E_PADX - N_EDGES))
    sw_pad = jnp.pad(self_w, (0, E_PADX - N_EDGES))

    h = _emb_bag(emb_split, idx_pad, offs_pad, input_bias)
    pp, rr = _gcn_scatter(h, src_pad, dst2d, ppi_pad, sw_pad)
    h = _node_update(pp, rr, W1.T, b1.reshape(1, HIDDEN))
    pp, rr = _gcn_scatter(h, src_pad, dst2d, ppi_pad, sw_pad)
    return _final_mm(pp, rr, W2.T, b2.reshape(1, HIDDEN),
                     Wout.T, bout.reshape(1, LABELS))


# speedup vs baseline: 34.9815x; 1.0645x over previous
import functools

import jax
import jax.numpy as jnp
from jax import lax
from jax.experimental import pallas as pl
from jax.experimental.pallas import tpu as pltpu
from jax.experimental.pallas import tpu_sc as plsc

N_NODES = 10000
N_EDGES = 320000
NNZ = 300000
INPUT_SIZE = 40000
HIDDEN = 128
LABELS = 2000
HALF = 64

NC = 2
NS = 16

CH_E = 128
KI = 148
NNZ_TILE = KI * CH_E
NNZ_PAD = NNZ_TILE * NS
OFFS_PAD = 10016

CH_G = 256
SUP = 4
SUPE = SUP * CH_G
NSUP = 20
E_TILE = NSUP * SUPE
E_PAD = E_TILE * NS
E_PADX = E_PAD + 2 * SUPE

ACC_ROWS = 10240
ACC_G = 12288
TRASH = N_NODES
WSTRIPE = N_NODES // NS

_mesh = plsc.VectorSubcoreMesh(core_axis_name="c", subcore_axis_name="s")
_params = pltpu.CompilerParams(use_tc_tiling_on_sc=False,
                               needs_layout_passes=False)


def _zero_rows(buf, n):
    w = 32 if buf.dtype == jnp.bfloat16 else 16
    z = jnp.zeros((w,), buf.dtype)
    cols = buf.shape[1]

    def zrow(r, _):
        for u in range(cols // w):
            buf[r, pl.ds(u * w, w)] = z
        return 0

    lax.fori_loop(0, n, zrow, 0)


@functools.partial(
    pl.kernel,
    out_type=jax.ShapeDtypeStruct((NC, N_NODES, HALF), jnp.bfloat16),
    mesh=_mesh,
    compiler_params=_params,
    scratch_types=[
        pltpu.VMEM((NNZ_TILE,), jnp.int32),
        pltpu.VMEM((4, CH_E), jnp.int32),
        pltpu.VMEM((OFFS_PAD,), jnp.int32),
        [pltpu.VMEM((CH_E, HALF), jnp.float32) for _ in range(4)],
        pltpu.VMEM((125, HALF), jnp.float32),
        pltpu.VMEM((125, HALF), jnp.bfloat16),
        pltpu.VMEM((HALF,), jnp.float32),
        pltpu.VMEM_SHARED((ACC_ROWS, HALF), jnp.float32),
        [pltpu.SemaphoreType.DMA for _ in range(4)],
        [pltpu.SemaphoreType.DMA for _ in range(4)],
    ],
)
def _emb_bag(emb_hbm, idx_hbm, offs_hbm, bias_hbm, out_hbm,
             idx_v, seg_v, offs_v, gbufs, obuf, obuf16, bias_v, acc,
             gsems, ssems):
    c = lax.axis_index("c")
    s = lax.axis_index("s")

    _zero_rows(gbufs[0], CH_E)
    _zero_rows(gbufs[1], CH_E)
    _zero_rows(gbufs[2], CH_E)
    _zero_rows(gbufs[3], CH_E)
    for t in range(5):
        pltpu.sync_copy(gbufs[t % 4], acc.at[pl.ds(s * 640 + t * CH_E, CH_E)])
    pltpu.sync_copy(offs_hbm, offs_v)
    pltpu.sync_copy(idx_hbm.at[pl.ds(s * NNZ_TILE, NNZ_TILE)], idx_v)
    pltpu.sync_copy(bias_hbm.at[pl.ds(c * HALF, HALF)], bias_v)

    @plsc.parallel_loop(0, NNZ_TILE // 16, unroll=8)
    def ixf(k):
        v = idx_v[pl.ds(k * 16, 16)]
        idx_v[pl.ds(k * 16, 16)] = v * 2 + c

    plsc.subcore_barrier()
    for b in range(4):
        pltpu.async_copy(gbufs[b], acc.at[pl.ds(10048, CH_E)], ssems[b])

    pbase = s * NNZ_TILE
    emb_c = emb_hbm

    def chunk_group(k4, _):
        for b in range(4):
            j = k4 * 4 + b
            pltpu.make_async_copy(
                gbufs[b], acc.at[pl.ds(0, CH_E)], ssems[b]).wait()
            pltpu.async_copy(emb_c.at[idx_v.at[pl.ds(j * CH_E, CH_E)]],
                             gbufs[b], gsems[b])
            ps = tuple(pbase + j * CH_E + u * 16 + lax.iota(jnp.int32, 16)
                       for u in range(8))
            lo0 = tuple(jnp.zeros((16,), jnp.int32) for _ in range(8))
            hi0 = tuple(jnp.full((16,), 10001, jnp.int32) for _ in range(8))

            def bs(_, lohi):
                los, his = lohi
                nlo, nhi = [], []
                for u in range(8):
                    mid = (los[u] + his[u]) >> 1
                    v = plsc.load_gather(offs_v, [mid])
                    cge = v <= ps[u]
                    nlo.append(jnp.where(cge, mid + 1, los[u]))
                    nhi.append(jnp.where(cge, his[u], mid))
                return (tuple(nlo), tuple(nhi))

            los, his = lax.fori_loop(0, 14, bs, (lo0, hi0))
            for u in range(8):
                seg_v[b, pl.ds(u * 16, 16)] = los[u] - 1
            pltpu.make_async_copy(
                emb_c.at[idx_v.at[pl.ds(j * CH_E, CH_E)]],
                gbufs[b], gsems[b]).wait()
            pltpu.async_copy(gbufs[b], acc.at[seg_v.at[b]], ssems[b],
                             add=True)
        return 0

    lax.fori_loop(0, KI // 4, chunk_group, 0)
    for b in range(4):
        pltpu.make_async_copy(
            gbufs[b], acc.at[pl.ds(0, CH_E)], ssems[b]).wait()
    plsc.subcore_barrier()

    ev = lax.iota(jnp.int32, 16) * 2
    for t in range(5):
        row0 = s * WSTRIPE + t * 125
        pltpu.sync_copy(acc.at[pl.ds(row0, 125)], obuf)

        @plsc.parallel_loop(0, 125, unroll=4)
        def orow(r):
            for u in range(4):
                x = obuf[r, pl.ds(u * 16, 16)] + bias_v[pl.ds(u * 16, 16)]
                obuf[r, pl.ds(u * 16, 16)] = jnp.maximum(x, 0.0)
            rr = jnp.full((16,), r, jnp.int32)
            for u in range(2):
                xa = plsc.load_gather(obuf, [rr, ev + u * 32])
                xb = plsc.load_gather(obuf, [rr, ev + u * 32 + 1])
                obuf16[r, pl.ds(u * 32, 32)] = plsc.pack(
                    xa, xb, format=plsc.PackFormat.INTERLEAVED)
        pltpu.sync_copy(obuf16, out_hbm.at[c, pl.ds(row0, 125)])


@functools.partial(
    pl.kernel,
    out_type=(jax.ShapeDtypeStruct((NC, N_NODES, HALF), jnp.bfloat16),
              jax.ShapeDtypeStruct((NC, N_NODES, HALF), jnp.bfloat16)),
    mesh=_mesh,
    compiler_params=_params,
    scratch_types=[
        pltpu.VMEM((2, SUPE), jnp.int32),
        pltpu.VMEM((2, SUP, CH_G), jnp.int32),
        pltpu.VMEM((2, SUPE), jnp.float32),
        pltpu.VMEM((2, SUPE), jnp.float32),
        [pltpu.VMEM((CH_G, HALF), jnp.bfloat16) for _ in range(2)],
        [pltpu.VMEM((CH_G, HIDDEN), jnp.bfloat16) for _ in range(2)],
        pltpu.VMEM((125, HALF), jnp.bfloat16),
        pltpu.VMEM_SHARED((ACC_G, HIDDEN), jnp.bfloat16),
        [pltpu.SemaphoreType.DMA for _ in range(2)],
        [pltpu.SemaphoreType.DMA for _ in range(2)],
        pltpu.SemaphoreType.DMA,
    ],
)
def _gcn_scatter(h_hbm, src_hbm, dst_hbm, ppi_hbm, sw_hbm, outp_hbm, outr_hbm,
                 src_v, dst_v, ppi_v, sw_v, gbufs, sbufs, obuf, acc,
                 gsems, ssems, msem):
    c = lax.axis_index("c")
    s = lax.axis_index("s")
    h_c = h_hbm.at[c]
    ebase = s * E_TILE
    dbase = s * (E_TILE // CH_G)

    def _meta_fire(g, slot):
        pltpu.async_copy(src_hbm.at[pl.ds(ebase + g * SUPE, SUPE)],
                         src_v.at[slot], msem)
        pltpu.async_copy(dst_hbm.at[pl.ds(dbase + g * SUP, SUP)],
                         dst_v.at[slot], msem)
        pltpu.async_copy(ppi_hbm.at[pl.ds(ebase + g * SUPE, SUPE)],
                         ppi_v.at[slot], msem)
        pltpu.async_copy(sw_hbm.at[pl.ds(ebase + g * SUPE, SUPE)],
                         sw_v.at[slot], msem)

    def _meta_wait(slot):
        pltpu.make_async_copy(src_hbm.at[pl.ds(0, SUPE)],
                              src_v.at[slot], msem).wait()
        pltpu.make_async_copy(dst_hbm.at[pl.ds(0, SUP)],
                              dst_v.at[slot], msem).wait()
        pltpu.make_async_copy(ppi_hbm.at[pl.ds(0, SUPE)],
                              ppi_v.at[slot], msem).wait()
        pltpu.make_async_copy(sw_hbm.at[pl.ds(0, SUPE)],
                              sw_v.at[slot], msem).wait()

    def _gather_fire(slot, jj, b):
        pltpu.async_copy(
            h_c.at[src_v.at[slot].at[pl.ds(jj * CH_G, CH_G)]],
            gbufs[b], gsems[b])

    def _gather_wait(b):
        pltpu.make_async_copy(h_c.at[src_v.at[0].at[pl.ds(0, CH_G)]],
                              gbufs[b], gsems[b]).wait()

    def _scatter_wait(b):
        pltpu.make_async_copy(sbufs[b], acc.at[pl.ds(0, CH_G)],
                              ssems[b]).wait()

    _zero_rows(sbufs[0], CH_G)
    _zero_rows(sbufs[1], CH_G)
    for t in range(2):
        pltpu.sync_copy(sbufs[t], acc.at[pl.ds(s * 640 + t * CH_G, CH_G)])
    pltpu.sync_copy(sbufs[0].at[pl.ds(0, 128)],
                    acc.at[pl.ds(s * 640 + 2 * CH_G, 128)])
    plsc.subcore_barrier()
    for b in range(2):
        pltpu.async_copy(sbufs[b], acc.at[pl.ds(10496 + b * CH_G, CH_G)],
                         ssems[b])

    _meta_fire(0, 0)
    _meta_wait(0)
    _gather_fire(0, 0, 0)

    def pair_body(i, _):
        for m in range(2):
            g = i * 2 + m
            for jj in range(SUP):
                b = jj % 2
                _gather_wait(b)
                if jj < SUP - 1:
                    _gather_fire(m, jj + 1, b ^ 1)
                else:
                    _meta_wait(m ^ 1)
                    _gather_fire(m ^ 1, 0, b ^ 1)
                _scatter_wait(b)
                if jj == 1:
                    _meta_fire(g + 1, m ^ 1)

                @plsc.parallel_loop(0, CH_G, unroll=4)
                def row(r):
                    e = jnp.full((16,), jj * CH_G, jnp.int32) + r
                    sm = jnp.full((16,), m, jnp.int32)
                    wpf = plsc.load_gather(ppi_v, [sm, e])
                    wsf = plsc.load_gather(sw_v, [sm, e])
                    wp = plsc.pack(wpf, wpf, format=plsc.PackFormat.INTERLEAVED)
                    ws = plsc.pack(wsf, wsf, format=plsc.PackFormat.INTERLEAVED)
                    for u in range(2):
                        x = gbufs[b][r, pl.ds(u * 32, 32)]
                        sbufs[b][r, pl.ds(u * 32, 32)] = x * wp
                        sbufs[b][r, pl.ds(HALF + u * 32, 32)] = x * ws
                pltpu.async_copy(sbufs[b], acc.at[dst_v.at[m, jj]],
                                 ssems[b], add=True)
        return 0

    lax.fori_loop(0, NSUP // 2, pair_body, 0)

    _gather_wait(0)
    _scatter_wait(0)
    _scatter_wait(1)
    plsc.subcore_barrier()

    for t in range(5):
        row0 = s * WSTRIPE + t * 125
        pltpu.sync_copy(acc.at[pl.ds(row0, 125), pl.ds(0, HALF)], obuf)
        pltpu.sync_copy(obuf, outp_hbm.at[c, pl.ds(row0, 125)])
        pltpu.sync_copy(acc.at[pl.ds(row0, 125), pl.ds(HALF, HALF)], obuf)
        pltpu.sync_copy(obuf, outr_hbm.at[c, pl.ds(row0, 125)])


_BM1 = 2000


def _node_update_body(p_ref, r_ref, w_ref, b_ref, o_ref):
    x = (jnp.dot(p_ref[0], w_ref[:HALF, :], preferred_element_type=jnp.float32)
         + jnp.dot(p_ref[1], w_ref[HALF:, :], preferred_element_type=jnp.float32))
    r = jnp.concatenate([r_ref[0], r_ref[1]], axis=-1)
    h = jnp.maximum(x + b_ref[0] + r, 0.0).astype(jnp.bfloat16)
    o_ref[0] = h[:, :HALF]
    o_ref[1] = h[:, HALF:]


_node_update = pl.pallas_call(
    _node_update_body,
    grid=(N_NODES // _BM1,),
    in_specs=[
        pl.BlockSpec((NC, _BM1, HALF), lambda i: (0, i, 0)),
        pl.BlockSpec((NC, _BM1, HALF), lambda i: (0, i, 0)),
        pl.BlockSpec((HIDDEN, HIDDEN), lambda i: (0, 0)),
        pl.BlockSpec((1, HIDDEN), lambda i: (0, 0)),
    ],
    out_specs=pl.BlockSpec((NC, _BM1, HALF), lambda i: (0, i, 0)),
    out_shape=jax.ShapeDtypeStruct((NC, N_NODES, HALF), jnp.bfloat16),
)

_BM2 = 1000


def _final_body(p_ref, r_ref, w_ref, b_ref, wo_ref, bo_ref, o_ref):
    x = (jnp.dot(p_ref[0], w_ref[:HALF, :], preferred_element_type=jnp.float32)
         + jnp.dot(p_ref[1], w_ref[HALF:, :], preferred_element_type=jnp.float32))
    r = jnp.concatenate([r_ref[0], r_ref[1]], axis=-1)
    h = jnp.maximum(x + b_ref[0] + r, 0.0)
    o_ref[...] = jnp.dot(h, wo_ref[...], preferred_element_type=jnp.float32) + bo_ref[0]


_final_mm = pl.pallas_call(
    _final_body,
    grid=(N_NODES // _BM2,),
    in_specs=[
        pl.BlockSpec((NC, _BM2, HALF), lambda i: (0, i, 0)),
        pl.BlockSpec((NC, _BM2, HALF), lambda i: (0, i, 0)),
        pl.BlockSpec((HIDDEN, HIDDEN), lambda i: (0, 0)),
        pl.BlockSpec((1, HIDDEN), lambda i: (0, 0)),
        pl.BlockSpec((HIDDEN, LABELS), lambda i: (0, 0)),
        pl.BlockSpec((1, LABELS), lambda i: (0, 0)),
    ],
    out_specs=pl.BlockSpec((_BM2, LABELS), lambda i: (i, 0)),
    out_shape=jax.ShapeDtypeStruct((N_NODES, LABELS), jnp.float32),
)


def kernel(indices, offsets, edge_index, ppi, self_w, emb_table, input_bias,
           W1, b1, W2, b2, Wout, bout):
    emb_split = emb_table.reshape(INPUT_SIZE * NC, HALF)
    idx_pad = jnp.pad(indices, (0, NNZ_PAD - NNZ))
    offs_pad = jnp.pad(offsets, (0, OFFS_PAD - (N_NODES + 1)),
                       constant_values=NNZ)
    src_pad = jnp.pad(edge_index[0], (0, E_PADX - N_EDGES))
    dst2d = jnp.pad(edge_index[1], (0, E_PADX - N_EDGES),
                    constant_values=TRASH).reshape(E_PADX // CH_G, CH_G)
    ppi_pad = jnp.pad(ppi, (0, E_PADX - N_EDGES))
    sw_pad = jnp.pad(self_w, (0, E_PADX - N_EDGES))

    h = _emb_bag(emb_split, idx_pad, offs_pad, input_bias)
    pp, rr = _gcn_scatter(h, src_pad, dst2d, ppi_pad, sw_pad)
    h = _node_update(pp, rr, W1.T, b1.reshape(1, HIDDEN))
    pp, rr = _gcn_scatter(h, src_pad, dst2d, ppi_pad, sw_pad)
    return _final_mm(pp, rr, W2.T, b2.reshape(1, HIDDEN),
                     Wout.T, bout.reshape(1, LABELS))


# speedup vs baseline: 36.0610x; 1.0309x over previous
import functools

import jax
import jax.numpy as jnp
from jax import lax
from jax.experimental import pallas as pl
from jax.experimental.pallas import tpu as pltpu
from jax.experimental.pallas import tpu_sc as plsc

N_NODES = 10000
N_EDGES = 320000
NNZ = 300000
INPUT_SIZE = 40000
HIDDEN = 128
LABELS = 2000
HALF = 64

NC = 2
NS = 16

CH_E = 256
KI = 74
NNZ_TILE = KI * CH_E
NNZ_PAD = NNZ_TILE * NS
OFFS_PAD = 10016

CH_G = 256
SUP = 4
SUPE = SUP * CH_G
NSUP = 20
E_TILE = NSUP * SUPE
E_PAD = E_TILE * NS
E_PADX = E_PAD + 2 * SUPE

ACC_ROWS = 10240
ACC_G = 12288
TRASH = N_NODES
WSTRIPE = N_NODES // NS

_mesh = plsc.VectorSubcoreMesh(core_axis_name="c", subcore_axis_name="s")
_params = pltpu.CompilerParams(use_tc_tiling_on_sc=False,
                               needs_layout_passes=False)


def _zero_rows(buf, n):
    w = 32 if buf.dtype == jnp.bfloat16 else 16
    z = jnp.zeros((w,), buf.dtype)
    cols = buf.shape[1]

    def zrow(r, _):
        for u in range(cols // w):
            buf[r, pl.ds(u * w, w)] = z
        return 0

    lax.fori_loop(0, n, zrow, 0)


@functools.partial(
    pl.kernel,
    out_type=jax.ShapeDtypeStruct((NC, N_NODES, HALF), jnp.bfloat16),
    mesh=_mesh,
    compiler_params=_params,
    scratch_types=[
        pltpu.VMEM((NNZ_TILE,), jnp.int32),
        pltpu.VMEM((2, CH_E), jnp.int32),
        pltpu.VMEM((OFFS_PAD,), jnp.int32),
        [pltpu.VMEM((CH_E, HALF), jnp.float32) for _ in range(2)],
        pltpu.VMEM((125, HALF), jnp.float32),
        pltpu.VMEM((125, HALF), jnp.bfloat16),
        pltpu.VMEM((HALF,), jnp.float32),
        pltpu.VMEM_SHARED((ACC_G, HALF), jnp.float32),
        [pltpu.SemaphoreType.DMA for _ in range(2)],
        [pltpu.SemaphoreType.DMA for _ in range(2)],
    ],
)
def _emb_bag(emb_hbm, idx_hbm, offs_hbm, bias_hbm, out_hbm,
             idx_v, seg_v, offs_v, gbufs, obuf, obuf16, bias_v, acc,
             gsems, ssems):
    c = lax.axis_index("c")
    s = lax.axis_index("s")

    _zero_rows(gbufs[0], CH_E)
    _zero_rows(gbufs[1], CH_E)
    for t in range(2):
        pltpu.sync_copy(gbufs[t], acc.at[pl.ds(s * 640 + t * CH_E, CH_E)])
    pltpu.sync_copy(gbufs[0].at[pl.ds(0, 128)],
                    acc.at[pl.ds(s * 640 + 2 * CH_E, 128)])
    pltpu.sync_copy(offs_hbm, offs_v)
    pltpu.sync_copy(idx_hbm.at[pl.ds(s * NNZ_TILE, NNZ_TILE)], idx_v)
    pltpu.sync_copy(bias_hbm.at[pl.ds(c * HALF, HALF)], bias_v)

    @plsc.parallel_loop(0, NNZ_TILE // 16, unroll=8)
    def ixf(k):
        v = idx_v[pl.ds(k * 16, 16)]
        idx_v[pl.ds(k * 16, 16)] = v * 2 + c

    plsc.subcore_barrier()
    for b in range(2):
        pltpu.async_copy(gbufs[b], acc.at[pl.ds(10496 + b * CH_E, CH_E)],
                         ssems[b])

    pbase = s * NNZ_TILE
    emb_c = emb_hbm

    def chunk_group(k2, _):
        for b in range(2):
            j = k2 * 2 + b
            pltpu.make_async_copy(
                gbufs[b], acc.at[pl.ds(0, CH_E)], ssems[b]).wait()
            pltpu.async_copy(emb_c.at[idx_v.at[pl.ds(j * CH_E, CH_E)]],
                             gbufs[b], gsems[b])
            ps = tuple(pbase + j * CH_E + u * 16 + lax.iota(jnp.int32, 16)
                       for u in range(16))
            lo0 = tuple(jnp.zeros((16,), jnp.int32) for _ in range(16))
            hi0 = tuple(jnp.full((16,), 10001, jnp.int32) for _ in range(16))

            def bs(_, lohi):
                los, his = lohi
                nlo, nhi = [], []
                for u in range(16):
                    mid = (los[u] + his[u]) >> 1
                    v = plsc.load_gather(offs_v, [mid])
                    cge = v <= ps[u]
                    nlo.append(jnp.where(cge, mid + 1, los[u]))
                    nhi.append(jnp.where(cge, his[u], mid))
                return (tuple(nlo), tuple(nhi))

            los, his = lax.fori_loop(0, 14, bs, (lo0, hi0))
            for u in range(16):
                seg_v[b, pl.ds(u * 16, 16)] = los[u] - 1
            pltpu.make_async_copy(
                emb_c.at[idx_v.at[pl.ds(j * CH_E, CH_E)]],
                gbufs[b], gsems[b]).wait()
            pltpu.async_copy(gbufs[b], acc.at[seg_v.at[b]], ssems[b],
                             add=True)
        return 0

    lax.fori_loop(0, KI // 2, chunk_group, 0)
    for b in range(2):
        pltpu.make_async_copy(
            gbufs[b], acc.at[pl.ds(0, CH_E)], ssems[b]).wait()
    plsc.subcore_barrier()

    ev = lax.iota(jnp.int32, 16) * 2
    for t in range(5):
        row0 = s * WSTRIPE + t * 125
        pltpu.sync_copy(acc.at[pl.ds(row0, 125)], obuf)

        @plsc.parallel_loop(0, 125, unroll=4)
        def orow(r):
            for u in range(4):
                x = obuf[r, pl.ds(u * 16, 16)] + bias_v[pl.ds(u * 16, 16)]
                obuf[r, pl.ds(u * 16, 16)] = jnp.maximum(x, 0.0)
            rr = jnp.full((16,), r, jnp.int32)
            for u in range(2):
                xa = plsc.load_gather(obuf, [rr, ev + u * 32])
                xb = plsc.load_gather(obuf, [rr, ev + u * 32 + 1])
                obuf16[r, pl.ds(u * 32, 32)] = plsc.pack(
                    xa, xb, format=plsc.PackFormat.INTERLEAVED)
        pltpu.sync_copy(obuf16, out_hbm.at[c, pl.ds(row0, 125)])


@functools.partial(
    pl.kernel,
    out_type=(jax.ShapeDtypeStruct((NC, N_NODES, HALF), jnp.bfloat16),
              jax.ShapeDtypeStruct((NC, N_NODES, HALF), jnp.bfloat16)),
    mesh=_mesh,
    compiler_params=_params,
    scratch_types=[
        pltpu.VMEM((2, SUPE), jnp.int32),
        pltpu.VMEM((2, SUP, CH_G), jnp.int32),
        pltpu.VMEM((2, SUPE), jnp.float32),
        pltpu.VMEM((2, SUPE), jnp.float32),
        [pltpu.VMEM((CH_G, HALF), jnp.bfloat16) for _ in range(2)],
        [pltpu.VMEM((CH_G, HIDDEN), jnp.bfloat16) for _ in range(2)],
        pltpu.VMEM((125, HALF), jnp.bfloat16),
        pltpu.VMEM_SHARED((ACC_G, HIDDEN), jnp.bfloat16),
        [pltpu.SemaphoreType.DMA for _ in range(2)],
        [pltpu.SemaphoreType.DMA for _ in range(2)],
        pltpu.SemaphoreType.DMA,
    ],
)
def _gcn_scatter(h_hbm, src_hbm, dst_hbm, ppi_hbm, sw_hbm, outp_hbm, outr_hbm,
                 src_v, dst_v, ppi_v, sw_v, gbufs, sbufs, obuf, acc,
                 gsems, ssems, msem):
    c = lax.axis_index("c")
    s = lax.axis_index("s")
    h_c = h_hbm.at[c]
    ebase = s * E_TILE
    dbase = s * (E_TILE // CH_G)

    def _meta_fire(g, slot):
        pltpu.async_copy(src_hbm.at[pl.ds(ebase + g * SUPE, SUPE)],
                         src_v.at[slot], msem)
        pltpu.async_copy(dst_hbm.at[pl.ds(dbase + g * SUP, SUP)],
                         dst_v.at[slot], msem)
        pltpu.async_copy(ppi_hbm.at[pl.ds(ebase + g * SUPE, SUPE)],
                         ppi_v.at[slot], msem)
        pltpu.async_copy(sw_hbm.at[pl.ds(ebase + g * SUPE, SUPE)],
                         sw_v.at[slot], msem)

    def _meta_wait(slot):
        pltpu.make_async_copy(src_hbm.at[pl.ds(0, SUPE)],
                              src_v.at[slot], msem).wait()
        pltpu.make_async_copy(dst_hbm.at[pl.ds(0, SUP)],
                              dst_v.at[slot], msem).wait()
        pltpu.make_async_copy(ppi_hbm.at[pl.ds(0, SUPE)],
                              ppi_v.at[slot], msem).wait()
        pltpu.make_async_copy(sw_hbm.at[pl.ds(0, SUPE)],
                              sw_v.at[slot], msem).wait()

    def _gather_fire(slot, jj, b):
        pltpu.async_copy(
            h_c.at[src_v.at[slot].at[pl.ds(jj * CH_G, CH_G)]],
            gbufs[b], gsems[b])

    def _gather_wait(b):
        pltpu.make_async_copy(h_c.at[src_v.at[0].at[pl.ds(0, CH_G)]],
                              gbufs[b], gsems[b]).wait()

    def _scatter_wait(b):
        pltpu.make_async_copy(sbufs[b], acc.at[pl.ds(0, CH_G)],
                              ssems[b]).wait()

    _zero_rows(sbufs[0], CH_G)
    _zero_rows(sbufs[1], CH_G)
    for t in range(2):
        pltpu.sync_copy(sbufs[t], acc.at[pl.ds(s * 640 + t * CH_G, CH_G)])
    pltpu.sync_copy(sbufs[0].at[pl.ds(0, 128)],
                    acc.at[pl.ds(s * 640 + 2 * CH_G, 128)])
    plsc.subcore_barrier()
    for b in range(2):
        pltpu.async_copy(sbufs[b], acc.at[pl.ds(10496 + b * CH_G, CH_G)],
                         ssems[b])

    _meta_fire(0, 0)
    _meta_wait(0)
    _gather_fire(0, 0, 0)

    def pair_body(i, _):
        for m in range(2):
            g = i * 2 + m
            for jj in range(SUP):
                b = jj % 2
                _gather_wait(b)
                if jj < SUP - 1:
                    _gather_fire(m, jj + 1, b ^ 1)
                else:
                    _meta_wait(m ^ 1)
                    _gather_fire(m ^ 1, 0, b ^ 1)
                _scatter_wait(b)
                if jj == 1:
                    _meta_fire(g + 1, m ^ 1)

                @plsc.parallel_loop(0, CH_G, unroll=4)
                def row(r):
                    e = jnp.full((16,), jj * CH_G, jnp.int32) + r
                    sm = jnp.full((16,), m, jnp.int32)
                    wpf = plsc.load_gather(ppi_v, [sm, e])
                    wsf = plsc.load_gather(sw_v, [sm, e])
                    wp = plsc.pack(wpf, wpf, format=plsc.PackFormat.INTERLEAVED)
                    ws = plsc.pack(wsf, wsf, format=plsc.PackFormat.INTERLEAVED)
                    for u in range(2):
                        x = gbufs[b][r, pl.ds(u * 32, 32)]
                        sbufs[b][r, pl.ds(u * 32, 32)] = x * wp
                        sbufs[b][r, pl.ds(HALF + u * 32, 32)] = x * ws
                pltpu.async_copy(sbufs[b], acc.at[dst_v.at[m, jj]],
                                 ssems[b], add=True)
        return 0

    lax.fori_loop(0, NSUP // 2, pair_body, 0)

    _gather_wait(0)
    _scatter_wait(0)
    _scatter_wait(1)
    plsc.subcore_barrier()

    for t in range(5):
        row0 = s * WSTRIPE + t * 125
        pltpu.sync_copy(acc.at[pl.ds(row0, 125), pl.ds(0, HALF)], obuf)
        pltpu.sync_copy(obuf, outp_hbm.at[c, pl.ds(row0, 125)])
        pltpu.sync_copy(acc.at[pl.ds(row0, 125), pl.ds(HALF, HALF)], obuf)
        pltpu.sync_copy(obuf, outr_hbm.at[c, pl.ds(row0, 125)])


_BM1 = 2000


def _node_update_body(p_ref, r_ref, w_ref, b_ref, o_ref):
    x = (jnp.dot(p_ref[0], w_ref[:HALF, :], preferred_element_type=jnp.float32)
         + jnp.dot(p_ref[1], w_ref[HALF:, :], preferred_element_type=jnp.float32))
    r = jnp.concatenate([r_ref[0], r_ref[1]], axis=-1)
    h = jnp.maximum(x + b_ref[0] + r, 0.0).astype(jnp.bfloat16)
    o_ref[0] = h[:, :HALF]
    o_ref[1] = h[:, HALF:]


_node_update = pl.pallas_call(
    _node_update_body,
    grid=(N_NODES // _BM1,),
    in_specs=[
        pl.BlockSpec((NC, _BM1, HALF), lambda i: (0, i, 0)),
        pl.BlockSpec((NC, _BM1, HALF), lambda i: (0, i, 0)),
        pl.BlockSpec((HIDDEN, HIDDEN), lambda i: (0, 0)),
        pl.BlockSpec((1, HIDDEN), lambda i: (0, 0)),
    ],
    out_specs=pl.BlockSpec((NC, _BM1, HALF), lambda i: (0, i, 0)),
    out_shape=jax.ShapeDtypeStruct((NC, N_NODES, HALF), jnp.bfloat16),
)

_BM2 = 1000


def _final_body(p_ref, r_ref, w_ref, b_ref, wo_ref, bo_ref, o_ref):
    x = (jnp.dot(p_ref[0], w_ref[:HALF, :], preferred_element_type=jnp.float32)
         + jnp.dot(p_ref[1], w_ref[HALF:, :], preferred_element_type=jnp.float32))
    r = jnp.concatenate([r_ref[0], r_ref[1]], axis=-1)
    h = jnp.maximum(x + b_ref[0] + r, 0.0)
    o_ref[...] = jnp.dot(h, wo_ref[...], preferred_element_type=jnp.float32) + bo_ref[0]


_final_mm = pl.pallas_call(
    _final_body,
    grid=(N_NODES // _BM2,),
    in_specs=[
        pl.BlockSpec((NC, _BM2, HALF), lambda i: (0, i, 0)),
        pl.BlockSpec((NC, _BM2, HALF), lambda i: (0, i, 0)),
        pl.BlockSpec((HIDDEN, HIDDEN), lambda i: (0, 0)),
        pl.BlockSpec((1, HIDDEN), lambda i: (0, 0)),
        pl.BlockSpec((HIDDEN, LABELS), lambda i: (0, 0)),
        pl.BlockSpec((1, LABELS), lambda i: (0, 0)),
    ],
    out_specs=pl.BlockSpec((_BM2, LABELS), lambda i: (i, 0)),
    out_shape=jax.ShapeDtypeStruct((N_NODES, LABELS), jnp.float32),
)


def kernel(indices, offsets, edge_index, ppi, self_w, emb_table, input_bias,
           W1, b1, W2, b2, Wout, bout):
    emb_split = emb_table.reshape(INPUT_SIZE * NC, HALF)
    idx_pad = jnp.pad(indices, (0, NNZ_PAD - NNZ))
    offs_pad = jnp.pad(offsets, (0, OFFS_PAD - (N_NODES + 1)),
                       constant_values=NNZ)
    src_pad = jnp.pad(edge_index[0], (0, E_PADX - N_EDGES))
    dst2d = jnp.pad(edge_index[1], (0, E_PADX - N_EDGES),
                    constant_values=TRASH).reshape(E_PADX // CH_G, CH_G)
    ppi_pad = jnp.pad(ppi, (0, E_PADX - N_EDGES))
    sw_pad = jnp.pad(self_w, (0, E_PADX - N_EDGES))

    h = _emb_bag(emb_split, idx_pad, offs_pad, input_bias)
    pp, rr = _gcn_scatter(h, src_pad, dst2d, ppi_pad, sw_pad)
    h = _node_update(pp, rr, W1.T, b1.reshape(1, HIDDEN))
    pp, rr = _gcn_scatter(h, src_pad, dst2d, ppi_pad, sw_pad)
    return _final_mm(pp, rr, W2.T, b2.reshape(1, HIDDEN),
                     Wout.T, bout.reshape(1, LABELS))
